# baseline (speedup 1.0000x reference)
import functools

import jax
import jax.numpy as jnp
from jax import lax
from jax.experimental import pallas as pl
from jax.experimental.pallas import tpu as pltpu
from jax.experimental.pallas import tpu_sc as plsc

RES = 128
P = 2
_N = [6 * (RES // (P ** k)) ** 2 for k in range(5)]

_SC_CORES = 2
_SC_SUBCORES = 16
_SC_TILES = _SC_CORES * _SC_SUBCORES


def _mm_body(x_ref, w_ref, b_ref, o_ref, *, relu):
    acc = jnp.dot(x_ref[...], w_ref[...], preferred_element_type=jnp.float32)
    if b_ref is not None:
        acc = acc + b_ref[...]
    if relu:
        acc = jnp.maximum(acc, 0.0)
    o_ref[...] = acc


def _mm(x, w, b=None, relu=False, bn=2048):
    n, ci = x.shape
    co = w.shape[1]
    bn = min(bn, n)
    grid = (n // bn,)
    in_specs = [
        pl.BlockSpec((bn, ci), lambda i: (i, 0)),
        pl.BlockSpec((ci, co), lambda i: (0, 0)),
    ]
    args = [x, w]
    if b is not None:
        in_specs.append(pl.BlockSpec((1, co), lambda i: (0, 0)))
        args.append(b.reshape(1, co))
    body = functools.partial(_mm_body, relu=relu)
    if b is None:
        body = lambda x_ref, w_ref, o_ref: _mm_body(x_ref, w_ref, None, o_ref, relu=relu)
    return pl.pallas_call(
        body,
        grid=grid,
        in_specs=in_specs,
        out_specs=pl.BlockSpec((bn, co), lambda i: (i, 0)),
        out_shape=jax.ShapeDtypeStruct((n, co), jnp.float32),
    )(*args)


def _mm2_body(a_ref, b_ref, wa_ref, wb_ref, o_ref):
    acc = jnp.dot(a_ref[...], wa_ref[...], preferred_element_type=jnp.float32)
    acc += jnp.dot(b_ref[...], wb_ref[...], preferred_element_type=jnp.float32)
    o_ref[...] = acc


def _mm2(a, b, wa, wb, bn=2048):
    n, ca = a.shape
    cb = b.shape[1]
    co = wa.shape[1]
    bn = min(bn, n)
    return pl.pallas_call(
        _mm2_body,
        grid=(n // bn,),
        in_specs=[
            pl.BlockSpec((bn, ca), lambda i: (i, 0)),
            pl.BlockSpec((bn, cb), lambda i: (i, 0)),
            pl.BlockSpec((ca, co), lambda i: (0, 0)),
            pl.BlockSpec((cb, co), lambda i: (0, 0)),
        ],
        out_specs=pl.BlockSpec((bn, co), lambda i: (i, 0)),
        out_shape=jax.ShapeDtypeStruct((n, co), jnp.float32),
    )(a, b, wa, wb)


def _combine_body(x_ref, ws_ref, wn_ref, p_ref, dp_ref, b_ref, o_ref, *, relu):
    deg = dp_ref[0, :, 0] + dp_ref[1, :, 0]
    invd = 1.0 / jnp.maximum(deg, 1.0)
    mean = (p_ref[0] + p_ref[1]) * invd[:, None]
    acc = jnp.dot(x_ref[...], ws_ref[...], preferred_element_type=jnp.float32)
    if wn_ref is not None:
        acc += jnp.dot(mean, wn_ref[...], preferred_element_type=jnp.float32)
    else:
        acc += mean
    acc += b_ref[...]
    if relu:
        acc = jnp.maximum(acc, 0.0)
    o_ref[...] = acc


def _combine(x, ws, wn, p, dp, b, relu, bn=2048):
    n, ci = x.shape
    co = ws.shape[1]
    w = p.shape[2]
    bn = min(bn, n)
    in_specs = [
        pl.BlockSpec((bn, ci), lambda i: (i, 0)),
        pl.BlockSpec((ci, co), lambda i: (0, 0)),
    ]
    args = [x, ws]
    if wn is not None:
        in_specs.append(pl.BlockSpec((ci, co), lambda i: (0, 0)))
        args.append(wn)
    in_specs += [
        pl.BlockSpec((2, bn, w), lambda i: (0, i, 0)),
        pl.BlockSpec((2, bn, 16), lambda i: (0, i, 0)),
        pl.BlockSpec((1, co), lambda i: (0, 0)),
    ]
    args += [p, dp, b.reshape(1, co)]

    if wn is not None:
        body = functools.partial(_combine_body, relu=relu)
    else:
        body = lambda x_ref, ws_ref, p_ref, dp_ref, b_ref, o_ref: _combine_body(
            x_ref, ws_ref, None, p_ref, dp_ref, b_ref, o_ref, relu=relu)
    return pl.pallas_call(
        body,
        grid=(n // bn,),
        in_specs=in_specs,
        out_specs=pl.BlockSpec((bn, co), lambda i: (i, 0)),
        out_shape=jax.ShapeDtypeStruct((n, co), jnp.float32),
    )(*args)


def _pool_body(x_ref, o_ref):
    o_ref[...] = jnp.mean(x_ref[...], axis=(1, 3))


def _pool(h, res):
    c = h.shape[1]
    r2 = res // 2
    m = 6 * r2
    x = h.reshape(m, 2, r2, 2, c)
    g = 8 if m % 8 == 0 else 1
    out = pl.pallas_call(
        _pool_body,
        grid=(m // g,),
        in_specs=[pl.BlockSpec((g, 2, r2, 2, c), lambda i: (i, 0, 0, 0, 0))],
        out_specs=pl.BlockSpec((g, r2, c), lambda i: (i, 0, 0)),
        out_shape=jax.ShapeDtypeStruct((m, r2, c), jnp.float32),
    )(x)
    return out.reshape(m * r2, c)


def _up(h, res, w, b):
    c, d = w.shape[0], w.shape[1]
    wr = w.transpose(0, 2, 3, 1).reshape(c, 4 * d)
    b4 = jnp.tile(b, 4)
    p = _mm(h, wr, b4)
    p = p.reshape(6, res, res, 2, 2, d).transpose(0, 1, 3, 2, 4, 5)
    return p.reshape(6 * 4 * res * res, d)


def _seg_group_size(e_tile):
    for g in range(min(128, e_tile), 0, -8):
        if e_tile % g == 0:
            return g
    raise ValueError(e_tile)


def _seg_config(e, n, w, n_passes):
    e_tile = e // _SC_TILES
    accum_bytes = (n // n_passes + 128) * w * 4
    g = _seg_group_size(e_tile)
    while True:
        ng = e_tile // g
        nc = max(d for d in range(1, min(ng, 32) + 1) if ng % d == 0)
        tile_bytes = 2 * nc * g * 4 + 4 * g * w * 4
        if accum_bytes + _SC_SUBCORES * tile_bytes <= int(7.6 * 1024 * 1024):
            return g, nc
        ng2 = g
        for g2 in range(g - 8, 0, -8):
            if e_tile % g2 == 0:
                ng2 = g2
                break
        if ng2 == g:
            return g, nc
        g = ng2


@functools.lru_cache(maxsize=None)
def _make_seg_kernel(n, e, w, n_passes, ones_mode):
    half = n // n_passes
    half_pad = half + 128
    zstripe = half // _SC_SUBCORES
    stripe = half // _SC_SUBCORES
    e_tile = e // _SC_TILES
    g, nc = _seg_config(e, n, w, n_passes)
    ng = e_tile // g
    n_chunks = ng // nc
    pipe = nc >= 2 and not ones_mode
    mesh = plsc.VectorSubcoreMesh(core_axis_name="c", subcore_axis_name="s")

    def body(*refs):
        if ones_mode:
            ones_hbm, *dstls, zeros_hbm, out_hbm, accum, dstv, rows = refs
            src_hbm = srcv = gsem = ssem = None
        else:
            (y_hbm, src_hbm, *dstls, zeros_hbm, out_hbm,
             accum, srcv, dstv, rows, gsem, ssem) = refs
        c = lax.axis_index("c")
        s = lax.axis_index("s")
        tid = c * _SC_SUBCORES + s
        if ones_mode:
            pltpu.sync_copy(ones_hbm, rows)
        for p in range(n_passes):
            pltpu.sync_copy(zeros_hbm, accum.at[pl.ds(s * zstripe, zstripe)])
            plsc.subcore_barrier()
            dstl = dstls[p]

            def chunkbody(ch, carry):
                gbase = tid * ng + ch * nc
                pltpu.sync_copy(dstl.at[pl.ds(gbase, nc)], dstv)
                if ones_mode:
                    def grp(j, cc):
                        pltpu.sync_copy(rows, accum.at[dstv.at[j]], add=True)
                        return cc
                    lax.fori_loop(0, nc, grp, 0)
                elif not pipe:
                    pltpu.sync_copy(src_hbm.at[pl.ds(gbase, nc)], srcv)

                    def grp(j, cc):
                        pltpu.async_copy(y_hbm.at[srcv.at[j]], rows.at[0],
                                         gsem.at[0]).wait()
                        pltpu.sync_copy(rows.at[0], accum.at[dstv.at[j]],
                                        add=True)
                        return cc
                    lax.fori_loop(0, nc, grp, 0)
                else:
                    pltpu.sync_copy(src_hbm.at[pl.ds(gbase, nc)], srcv)
                    for b in range(min(2, nc)):
                        pltpu.async_copy(y_hbm.at[srcv.at[b]], rows.at[b],
                                         gsem.at[b])

                    def quadbody(i, cc):
                        for b in range(4):
                            j = 4 * i + b

                            bf = (b + 2) % 4

                            @pl.when(j < nc)
                            def _():
                                pltpu.make_async_copy(
                                    y_hbm.at[srcv.at[0]], rows.at[b],
                                    gsem.at[b]).wait()
                                pltpu.async_copy(rows.at[b],
                                                 accum.at[dstv.at[j]],
                                                 ssem.at[b], add=True)

                                @pl.when(j >= 2)
                                def _():
                                    pltpu.make_async_copy(
                                        rows.at[bf], accum.at[dstv.at[0]],
                                        ssem.at[bf]).wait()

                                @pl.when(j + 2 < nc)
                                def _():
                                    pltpu.async_copy(y_hbm.at[srcv.at[j + 2]],
                                                     rows.at[bf], gsem.at[bf])
                        return cc

                    lax.fori_loop(0, (nc + 3) // 4, quadbody, 0)
                    for jt in range(max(0, nc - 2), nc):
                        b = jt % 4
                        pltpu.make_async_copy(rows.at[b], accum.at[dstv.at[0]],
                                              ssem.at[b]).wait()
                return carry

            lax.fori_loop(0, n_chunks, chunkbody, 0)
            plsc.subcore_barrier()
            pltpu.sync_copy(
                accum.at[pl.ds(s * stripe, stripe)],
                out_hbm.at[pl.ds(c * n + p * half + s * stripe, stripe)])
            if p + 1 < n_passes:
                plsc.subcore_barrier()

    scratch = [pltpu.VMEM_SHARED((half_pad, w), jnp.float32)]
    if not ones_mode:
        scratch.append(pltpu.VMEM((nc, g), jnp.int32))
    scratch.append(pltpu.VMEM((nc, g), jnp.int32))
    if ones_mode:
        scratch.append(pltpu.VMEM((g, w), jnp.float32))
    else:
        nbuf = 4 if pipe else 1
        scratch += [
            pltpu.VMEM((nbuf, g, w), jnp.float32),
            pltpu.SemaphoreType.DMA((nbuf,)),
            pltpu.SemaphoreType.DMA((nbuf,)),
        ]
    return pl.kernel(
        body,
        out_type=jax.ShapeDtypeStruct((_SC_CORES * n, w), jnp.float32),
        mesh=mesh,
        scratch_types=scratch,
        compiler_params=pltpu.CompilerParams(use_tc_tiling_on_sc=False),
    )


def _seg_partials(y, src, dstls, n):
    w = y.shape[1]
    e = src.shape[0]
    g, _ = _seg_config(e, n, w, len(dstls))
    zeros = jnp.zeros((n // len(dstls) // _SC_SUBCORES, w), jnp.float32)
    kfn = _make_seg_kernel(n, e, w, len(dstls), False)
    out = kfn(y, src.reshape(e // g, g), *[d.reshape(e // g, g) for d in dstls],
              zeros)
    return out.reshape(_SC_CORES, n, w)


def _deg_partials(dstls, e, n):
    g, _ = _seg_config(e, n, 16, len(dstls))
    ones = jnp.ones((g, 16), jnp.float32)
    zeros = jnp.zeros((n // len(dstls) // _SC_SUBCORES, 16), jnp.float32)
    kfn = _make_seg_kernel(n, e, 16, len(dstls), True)
    out = kfn(ones, *[d.reshape(e // g, g) for d in dstls], zeros)
    return out.reshape(_SC_CORES, n, 16)


_PNC = 8


@functools.lru_cache(maxsize=None)
def _make_partition_kernel(n, e):
    half = n // 2
    e_tile = e // _SC_TILES
    g = 128
    capg = ((e_tile // g + _PNC) // _PNC + 1) * _PNC
    cap = capg * g
    cbuf = 2048
    nch = e_tile // cbuf
    pad_unit = _PNC * g
    mesh = plsc.VectorSubcoreMesh(core_axis_name="c", subcore_axis_name="s")

    def body(src_hbm, dst_hbm, psrc_hbm, pdst_hbm, cnt_hbm,
             in_src, in_dst, lo_src, lo_dst, hi_src, hi_dst, cnt_v):
        c = lax.axis_index("c")
        s = lax.axis_index("s")
        tid = c * _SC_SUBCORES + s
        ebase = tid * e_tile

        def chunk(ch, offs):
            base = ebase + ch * cbuf
            pltpu.sync_copy(src_hbm.at[pl.ds(base, cbuf)], in_src)
            pltpu.sync_copy(dst_hbm.at[pl.ds(base, cbuf)], in_dst)

            def vec(v, offs2):
                off_lo, off_hi = offs2
                sv = in_src[pl.ds(v * 16, 16)]
                dv = in_dst[pl.ds(v * 16, 16)]
                mlo = dv < half
                nlo = jnp.max(plsc.all_reduce_population_count(mlo))
                dk, sp = plsc.sort_key_val(dv, sv)
                lo_src[pl.ds(off_lo, 16)] = sp
                lo_dst[pl.ds(off_lo, 16)] = dk
                dk2, sp2 = plsc.sort_key_val(dv, sv, descending=True)
                hi_src[pl.ds(off_hi, 16)] = sp2
                hi_dst[pl.ds(off_hi, 16)] = dk2 - half
                return off_lo + nlo, off_hi + (16 - nlo)

            return lax.fori_loop(0, cbuf // 16, vec, offs)

        off_lo, off_hi = lax.fori_loop(0, nch, chunk, (0, 0))

        zeros16 = jnp.zeros((16,), jnp.int32)
        garb16 = jnp.full((16,), half, jnp.int32)
        for p, (off, sbuf, dbuf) in enumerate(
                [(off_lo, lo_src, lo_dst), (off_hi, hi_src, hi_dst)]):
            npad = (pad_unit - off % pad_unit) % pad_unit

            def padv(k, _):
                sbuf[pl.ds(off + k * 16, 16)] = zeros16
                dbuf[pl.ds(off + k * 16, 16)] = garb16
                return 0

            lax.fori_loop(0, (npad + 15) // 16, padv, 0)
            total = off + npad
            pltpu.sync_copy(sbuf.at[pl.ds(0, cap)],
                            psrc_hbm.at[pl.ds((p * _SC_TILES + tid) * cap,
                                              cap)])
            pltpu.sync_copy(dbuf.at[pl.ds(0, cap)],
                            pdst_hbm.at[pl.ds((p * _SC_TILES + tid) * cap,
                                              cap)])
            cnt_v[...] = jnp.full((16,), total // pad_unit, jnp.int32)
            pltpu.sync_copy(cnt_v, cnt_hbm.at[p * _SC_TILES + tid])

    buf = cap + 32
    return pl.kernel(
        body,
        out_type=(
            jax.ShapeDtypeStruct((2 * _SC_TILES * cap,), jnp.int32),
            jax.ShapeDtypeStruct((2 * _SC_TILES * cap,), jnp.int32),
            jax.ShapeDtypeStruct((2 * _SC_TILES, 16), jnp.int32),
        ),
        mesh=mesh,
        scratch_types=[
            pltpu.VMEM((cbuf,), jnp.int32),
            pltpu.VMEM((cbuf,), jnp.int32),
            pltpu.VMEM((buf,), jnp.int32),
            pltpu.VMEM((buf,), jnp.int32),
            pltpu.VMEM((buf,), jnp.int32),
            pltpu.VMEM((buf,), jnp.int32),
            pltpu.VMEM((16,), jnp.int32),
        ],
        compiler_params=pltpu.CompilerParams(use_tc_tiling_on_sc=False,
                                             needs_layout_passes=False),
    )


def _partition_l0(src, dst, n):
    e = src.shape[0]
    e_tile = e // _SC_TILES
    capg = ((e_tile // 128 + _PNC) // _PNC + 1) * _PNC
    kfn = _make_partition_kernel(n, e)
    psrc, pdst, cnt = kfn(src, dst)
    return (psrc.reshape(2 * _SC_TILES * capg, 128),
            pdst.reshape(2 * _SC_TILES * capg, 128), cnt, capg)


@functools.lru_cache(maxsize=None)
def _make_seg_part_kernel(n, w, capg):
    half = n // 2
    half_pad = half + 128
    stripe = half // _SC_SUBCORES
    g = 128
    nc = _PNC
    mesh = plsc.VectorSubcoreMesh(core_axis_name="c", subcore_axis_name="s")

    def body(y_hbm, psrc_hbm, pdst_hbm, cnt_hbm, zeros_hbm, out_hbm,
             accum, srcv, dstv, cntv, rows, gsem, ssem):
        c = lax.axis_index("c")
        s = lax.axis_index("s")
        tid = c * _SC_SUBCORES + s
        for p in range(2):
            pltpu.sync_copy(zeros_hbm, accum.at[pl.ds(s * stripe, stripe)])
            pltpu.sync_copy(cnt_hbm.at[p * _SC_TILES + tid], cntv)
            plsc.subcore_barrier()
            nch = jnp.max(cntv[...])
            rowbase = (p * _SC_TILES + tid) * capg

            def chunkbody(ch, carry):
                gbase = rowbase + ch * nc
                pltpu.sync_copy(pdst_hbm.at[pl.ds(gbase, nc)], dstv)
                pltpu.sync_copy(psrc_hbm.at[pl.ds(gbase, nc)], srcv)
                for b in range(2):
                    pltpu.async_copy(y_hbm.at[srcv.at[b]], rows.at[b],
                                     gsem.at[b])

                def quadbody(i, cc):
                    for b in range(4):
                        j = 4 * i + b
                        bf = (b + 2) % 4

                        @pl.when(j < nc)
                        def _():
                            pltpu.make_async_copy(
                                y_hbm.at[srcv.at[0]], rows.at[b],
                                gsem.at[b]).wait()
                            pltpu.async_copy(rows.at[b], accum.at[dstv.at[j]],
                                             ssem.at[b], add=True)

                            @pl.when(j >= 2)
                            def _():
                                pltpu.make_async_copy(
                                    rows.at[bf], accum.at[dstv.at[0]],
                                    ssem.at[bf]).wait()

                            @pl.when(j + 2 < nc)
                            def _():
                                pltpu.async_copy(y_hbm.at[srcv.at[j + 2]],
                                                 rows.at[bf], gsem.at[bf])
                    return cc

                lax.fori_loop(0, (nc + 3) // 4, quadbody, 0)
                for jt in range(max(0, nc - 2), nc):
                    b = jt % 4
                    pltpu.make_async_copy(rows.at[b], accum.at[dstv.at[0]],
                                          ssem.at[b]).wait()
                return carry

            lax.fori_loop(0, nch, chunkbody, 0)
            plsc.subcore_barrier()
            pltpu.sync_copy(
                accum.at[pl.ds(s * stripe, stripe)],
                out_hbm.at[pl.ds(c * n + p * half + s * stripe, stripe)])
            if p == 0:
                plsc.subcore_barrier()

    return pl.kernel(
        body,
        out_type=jax.ShapeDtypeStruct((_SC_CORES * n, w), jnp.float32),
        mesh=mesh,
        scratch_types=[
            pltpu.VMEM_SHARED((half_pad, w), jnp.float32),
            pltpu.VMEM((nc, g), jnp.int32),
            pltpu.VMEM((nc, g), jnp.int32),
            pltpu.VMEM((16,), jnp.int32),
            pltpu.VMEM((4, g, w), jnp.float32),
            pltpu.SemaphoreType.DMA((4,)),
            pltpu.SemaphoreType.DMA((4,)),
        ],
        compiler_params=pltpu.CompilerParams(use_tc_tiling_on_sc=False,
                                             needs_layout_passes=False),
    )


def _seg_partials_part(y, part, n):
    psrc, pdst, cnt, capg = part
    w = y.shape[1]
    zeros = jnp.zeros((n // 2 // _SC_SUBCORES, w), jnp.float32)
    kfn = _make_seg_part_kernel(n, w, capg)
    out = kfn(y, psrc, pdst, cnt, zeros)
    return out.reshape(_SC_CORES, n, w)


def _dst_split_body(d_ref, lo_ref, hi_ref, *, half):
    d = d_ref[...]
    lo_ref[...] = jnp.where(d < half, d, half)
    hi_ref[...] = jnp.where(d >= half, d - half, half)


def _dst_split(dst, half):
    e = dst.shape[0]
    rows = e // 128
    x = dst.reshape(rows, 128)
    br = 512
    body = functools.partial(_dst_split_body, half=half)
    lo, hi = pl.pallas_call(
        body,
        grid=(rows // br,),
        in_specs=[pl.BlockSpec((br, 128), lambda i: (i, 0))],
        out_specs=[pl.BlockSpec((br, 128), lambda i: (i, 0))] * 2,
        out_shape=[jax.ShapeDtypeStruct((rows, 128), jnp.int32)] * 2,
    )(x)
    return lo.reshape(e), hi.reshape(e)


_SPMEM_BUDGET = 7 * 1024 * 1024


def _num_passes(n, w):
    p = 1
    while (n // p + 128) * w * 4 > _SPMEM_BUDGET:
        p *= 2
    return p


def _sage(x, src, dstls_fn, dp, params, name, n, relu=True, part=None):
    ws = params[name + "_Ws"]
    wn = params[name + "_Wn"]
    b = params[name + "_b"]
    ci, co = ws.shape
    w = min(ci, co)
    npass = _num_passes(n, w)

    def agg(y):
        if part is not None and npass == 2:
            return _seg_partials_part(y, part, n)
        return _seg_partials(y, src, dstls_fn(npass), n)

    if co < ci:
        y = _mm(x, wn)
        p = agg(y)
        return _combine(x, ws, None, p, dp, b, relu)
    p = agg(x)
    return _combine(x, ws, wn, p, dp, b, relu)


def kernel(in_feat, exteraVar1, params, edge_index1, edge_index2, edge_index3,
           edge_index4, edge_index5):
    del exteraVar1
    edges = [edge_index1, edge_index2, edge_index3, edge_index4, edge_index5]
    srcs = [e[0] for e in edges]
    dsts = [e[1] for e in edges]
    split_cache = {}

    def dstls_fn(lvl):
        def get(n_passes):
            if n_passes == 1:
                return [dsts[lvl]]
            key = (lvl, n_passes)
            if key not in split_cache:
                assert n_passes == 2
                split_cache[key] = list(_dst_split(dsts[lvl], _N[lvl] // 2))
            return split_cache[key]
        return get

    fns = [dstls_fn(l) for l in range(5)]
    dps = [_deg_partials(fns[l](_num_passes(_N[l], 16)), dsts[l].shape[0], _N[l])
           for l in range(5)]
    part0 = _partition_l0(srcs[0], dsts[0], _N[0])

    def sage(x, lvl, name, relu=True):
        return _sage(x, srcs[lvl], fns[lvl], dps[lvl], params, name, _N[lvl],
                     relu, part=part0 if lvl == 0 else None)

    h1 = sage(in_feat, 0, "conv1")
    h22 = sage(h1, 0, "conv2")
    h2 = _pool(h22, RES)
    h3 = sage(h2, 1, "conv3")
    h33 = sage(h3, 1, "conv33")
    h3p = _pool(h33, RES // 2)
    h4 = sage(h3p, 2, "conv4")
    h44 = sage(h4, 2, "conv44")
    h4p = _pool(h44, RES // 4)
    h5 = sage(h4p, 3, "conv5")
    h55 = sage(h5, 3, "conv55")
    h5p = _pool(h55, RES // 8)
    h6 = sage(h5p, 4, "conv6")
    h6 = sage(h6, 4, "conv66")
    h6 = sage(h6, 4, "conv7")
    h6 = _up(h6, RES // 16, params["up1_W"], params["up1_b"])
    h6 = jnp.concatenate([h6, h55], axis=1)
    h6 = sage(h6, 3, "conv7")
    h6 = sage(h6, 3, "conv77")
    h6 = sage(h6, 3, "conv8")
    h6 = _up(h6, RES // 8, params["up2_W"], params["up2_b"])
    h6 = jnp.concatenate([h6, h44], axis=1)
    h6 = sage(h6, 2, "conv8")
    h6 = sage(h6, 2, "conv88")
    h6 = sage(h6, 2, "conv9")
    h6 = _up(h6, RES // 4, params["up3_W"], params["up3_b"])
    h6 = jnp.concatenate([h6, h33], axis=1)
    h6 = sage(h6, 1, "conv9")
    h6 = sage(h6, 1, "conv99")
    h6 = sage(h6, 1, "conv10")
    h6 = _up(h6, RES // 2, params["up4_W"], params["up4_b"])
    h6 = jnp.concatenate([h6, h22], axis=1)
    h6 = sage(h6, 0, "conv10")
    h6 = sage(h6, 0, "conv101")
    return sage(h6, 0, "conv11", relu=False)


# speedup vs baseline: 1.0548x; 1.0548x over previous
import functools

import jax
import jax.numpy as jnp
from jax import lax
from jax.experimental import pallas as pl
from jax.experimental.pallas import tpu as pltpu
from jax.experimental.pallas import tpu_sc as plsc

RES = 128
P = 2
_N = [6 * (RES // (P ** k)) ** 2 for k in range(5)]

_SC_CORES = 2
_SC_SUBCORES = 16
_SC_TILES = _SC_CORES * _SC_SUBCORES


def _mm_body(x_ref, w_ref, b_ref, o_ref, *, relu):
    acc = jnp.dot(x_ref[...], w_ref[...], preferred_element_type=jnp.float32)
    if b_ref is not None:
        acc = acc + b_ref[...]
    if relu:
        acc = jnp.maximum(acc, 0.0)
    o_ref[...] = acc


def _mm(x, w, b=None, relu=False, bn=2048):
    n, ci = x.shape
    co = w.shape[1]
    bn = min(bn, n)
    grid = (n // bn,)
    in_specs = [
        pl.BlockSpec((bn, ci), lambda i: (i, 0)),
        pl.BlockSpec((ci, co), lambda i: (0, 0)),
    ]
    args = [x, w]
    if b is not None:
        in_specs.append(pl.BlockSpec((1, co), lambda i: (0, 0)))
        args.append(b.reshape(1, co))
    body = functools.partial(_mm_body, relu=relu)
    if b is None:
        body = lambda x_ref, w_ref, o_ref: _mm_body(x_ref, w_ref, None, o_ref, relu=relu)
    return pl.pallas_call(
        body,
        grid=grid,
        in_specs=in_specs,
        out_specs=pl.BlockSpec((bn, co), lambda i: (i, 0)),
        out_shape=jax.ShapeDtypeStruct((n, co), jnp.float32),
    )(*args)


def _mm2_body(a_ref, b_ref, wa_ref, wb_ref, o_ref):
    acc = jnp.dot(a_ref[...], wa_ref[...], preferred_element_type=jnp.float32)
    acc += jnp.dot(b_ref[...], wb_ref[...], preferred_element_type=jnp.float32)
    o_ref[...] = acc


def _mm2(a, b, wa, wb, bn=2048):
    n, ca = a.shape
    cb = b.shape[1]
    co = wa.shape[1]
    bn = min(bn, n)
    return pl.pallas_call(
        _mm2_body,
        grid=(n // bn,),
        in_specs=[
            pl.BlockSpec((bn, ca), lambda i: (i, 0)),
            pl.BlockSpec((bn, cb), lambda i: (i, 0)),
            pl.BlockSpec((ca, co), lambda i: (0, 0)),
            pl.BlockSpec((cb, co), lambda i: (0, 0)),
        ],
        out_specs=pl.BlockSpec((bn, co), lambda i: (i, 0)),
        out_shape=jax.ShapeDtypeStruct((n, co), jnp.float32),
    )(a, b, wa, wb)


def _unpack_rows(a, rows, w):
    k = 128 // w
    if k == 1:
        return a
    parts = [a[:, i * w:(i + 1) * w][:, None, :] for i in range(k)]
    return jnp.concatenate(parts, axis=1).reshape(rows, w)


def _combine_body(x_ref, ws_ref, wn_ref, p_ref, dp_ref, b_ref, o_ref, *, relu,
                  bn, w):
    pv = p_ref[...]
    if w <= 128:
        mean_raw = _unpack_rows(pv[0] + pv[1], bn, w)
    else:
        mean_raw = pv[0] + pv[1]
    dv = dp_ref[...]
    deg = _unpack_rows(dv[0] + dv[1], bn, 16)[:, 0]
    invd = 1.0 / jnp.maximum(deg, 1.0)
    mean = mean_raw * invd[:, None]
    acc = jnp.dot(x_ref[...], ws_ref[...], preferred_element_type=jnp.float32)
    if wn_ref is not None:
        acc += jnp.dot(mean, wn_ref[...], preferred_element_type=jnp.float32)
    else:
        acc += mean
    acc += b_ref[...]
    if relu:
        acc = jnp.maximum(acc, 0.0)
    o_ref[...] = acc


def _combine(x, ws, wn, p4, dp4, b, relu, w, bn=2048):
    n, ci = x.shape
    co = ws.shape[1]
    bn = min(bn, n)
    in_specs = [
        pl.BlockSpec((bn, ci), lambda i: (i, 0)),
        pl.BlockSpec((ci, co), lambda i: (0, 0)),
    ]
    args = [x, ws]
    if wn is not None:
        in_specs.append(pl.BlockSpec((ci, co), lambda i: (0, 0)))
        args.append(wn)
    if w <= 128:
        in_specs.append(pl.BlockSpec((2, bn * w // 128, 128),
                                     lambda i: (0, i, 0)))
    else:
        in_specs.append(pl.BlockSpec((2, bn, w), lambda i: (0, i, 0)))
    in_specs += [
        pl.BlockSpec((2, bn // 8, 128), lambda i: (0, i, 0)),
        pl.BlockSpec((1, co), lambda i: (0, 0)),
    ]
    args += [p4, dp4, b.reshape(1, co)]

    if wn is not None:
        body = functools.partial(_combine_body, relu=relu, bn=bn, w=w)
    else:
        body = lambda x_ref, ws_ref, p_ref, dp_ref, b_ref, o_ref: _combine_body(
            x_ref, ws_ref, None, p_ref, dp_ref, b_ref, o_ref, relu=relu,
            bn=bn, w=w)
    return pl.pallas_call(
        body,
        grid=(n // bn,),
        in_specs=in_specs,
        out_specs=pl.BlockSpec((bn, co), lambda i: (i, 0)),
        out_shape=jax.ShapeDtypeStruct((n, co), jnp.float32),
    )(*args)


def _pool_body(x_ref, o_ref):
    o_ref[...] = jnp.mean(x_ref[...], axis=(1, 3))


def _pool(h, res):
    c = h.shape[1]
    r2 = res // 2
    m = 6 * r2
    x = h.reshape(m, 2, r2, 2, c)
    g = 8 if m % 8 == 0 else 1
    out = pl.pallas_call(
        _pool_body,
        grid=(m // g,),
        in_specs=[pl.BlockSpec((g, 2, r2, 2, c), lambda i: (i, 0, 0, 0, 0))],
        out_specs=pl.BlockSpec((g, r2, c), lambda i: (i, 0, 0)),
        out_shape=jax.ShapeDtypeStruct((m, r2, c), jnp.float32),
    )(x)
    return out.reshape(m * r2, c)


def _up(h, res, w, b):
    c, d = w.shape[0], w.shape[1]
    wr = w.transpose(0, 2, 3, 1).reshape(c, 4 * d)
    b4 = jnp.tile(b, 4)
    p = _mm(h, wr, b4)
    p = p.reshape(6, res, res, 2, 2, d).transpose(0, 1, 3, 2, 4, 5)
    return p.reshape(6 * 4 * res * res, d)


def _seg_group_size(e_tile):
    for g in range(min(128, e_tile), 0, -8):
        if e_tile % g == 0:
            return g
    raise ValueError(e_tile)


def _seg_config(e, n, w, n_passes):
    e_tile = e // _SC_TILES
    accum_bytes = (n // n_passes + 128) * w * 4
    g = _seg_group_size(e_tile)
    while True:
        ng = e_tile // g
        nc = max(d for d in range(1, min(ng, 32) + 1) if ng % d == 0)
        tile_bytes = 2 * nc * g * 4 + 4 * g * w * 4
        if accum_bytes + _SC_SUBCORES * tile_bytes <= int(7.6 * 1024 * 1024):
            return g, nc
        ng2 = g
        for g2 in range(g - 8, 0, -8):
            if e_tile % g2 == 0:
                ng2 = g2
                break
        if ng2 == g:
            return g, nc
        g = ng2


@functools.lru_cache(maxsize=None)
def _make_seg_kernel(n, e, w, n_passes, ones_mode):
    half = n // n_passes
    half_pad = half + 128
    zstripe = half // _SC_SUBCORES
    stripe = half // _SC_SUBCORES
    e_tile = e // _SC_TILES
    g, nc = _seg_config(e, n, w, n_passes)
    ng = e_tile // g
    n_chunks = ng // nc
    pipe = nc >= 2 and not ones_mode
    mesh = plsc.VectorSubcoreMesh(core_axis_name="c", subcore_axis_name="s")

    def body(*refs):
        if ones_mode:
            ones_hbm, *dstls, zeros_hbm, out_hbm, accum, dstv, rows = refs
            src_hbm = srcv = gsem = ssem = None
        else:
            (y_hbm, src_hbm, *dstls, zeros_hbm, out_hbm,
             accum, srcv, dstv, rows, gsem, ssem) = refs
        c = lax.axis_index("c")
        s = lax.axis_index("s")
        tid = c * _SC_SUBCORES + s
        if ones_mode:
            pltpu.sync_copy(ones_hbm, rows)
        for p in range(n_passes):
            pltpu.sync_copy(zeros_hbm, accum.at[pl.ds(s * zstripe, zstripe)])
            plsc.subcore_barrier()
            dstl = dstls[p]

            def chunkbody(ch, carry):
                gbase = tid * ng + ch * nc
                pltpu.sync_copy(dstl.at[pl.ds(gbase, nc)], dstv)
                if ones_mode:
                    def grp(j, cc):
                        pltpu.sync_copy(rows, accum.at[dstv.at[j]], add=True)
                        return cc
                    lax.fori_loop(0, nc, grp, 0)
                elif not pipe:
                    pltpu.sync_copy(src_hbm.at[pl.ds(gbase, nc)], srcv)

                    def grp(j, cc):
                        pltpu.async_copy(y_hbm.at[srcv.at[j]], rows.at[0],
                                         gsem.at[0]).wait()
                        pltpu.sync_copy(rows.at[0], accum.at[dstv.at[j]],
                                        add=True)
                        return cc
                    lax.fori_loop(0, nc, grp, 0)
                else:
                    pltpu.sync_copy(src_hbm.at[pl.ds(gbase, nc)], srcv)
                    for b in range(min(2, nc)):
                        pltpu.async_copy(y_hbm.at[srcv.at[b]], rows.at[b],
                                         gsem.at[b])

                    def quadbody(i, cc):
                        for b in range(4):
                            j = 4 * i + b

                            bf = (b + 2) % 4

                            @pl.when(j < nc)
                            def _():
                                pltpu.make_async_copy(
                                    y_hbm.at[srcv.at[0]], rows.at[b],
                                    gsem.at[b]).wait()
                                pltpu.async_copy(rows.at[b],
                                                 accum.at[dstv.at[j]],
                                                 ssem.at[b], add=True)

                                @pl.when(j >= 2)
                                def _():
                                    pltpu.make_async_copy(
                                        rows.at[bf], accum.at[dstv.at[0]],
                                        ssem.at[bf]).wait()

                                @pl.when(j + 2 < nc)
                                def _():
                                    pltpu.async_copy(y_hbm.at[srcv.at[j + 2]],
                                                     rows.at[bf], gsem.at[bf])
                        return cc

                    lax.fori_loop(0, (nc + 3) // 4, quadbody, 0)
                    for jt in range(max(0, nc - 2), nc):
                        b = jt % 4
                        pltpu.make_async_copy(rows.at[b], accum.at[dstv.at[0]],
                                              ssem.at[b]).wait()
                return carry

            lax.fori_loop(0, n_chunks, chunkbody, 0)
            plsc.subcore_barrier()
            pltpu.sync_copy(
                accum.at[pl.ds(s * stripe, stripe)],
                out_hbm.at[pl.ds(c * n + p * half + s * stripe, stripe)])
            if p + 1 < n_passes:
                plsc.subcore_barrier()

    scratch = [pltpu.VMEM_SHARED((half_pad, w), jnp.float32)]
    if not ones_mode:
        scratch.append(pltpu.VMEM((nc, g), jnp.int32))
    scratch.append(pltpu.VMEM((nc, g), jnp.int32))
    if ones_mode:
        scratch.append(pltpu.VMEM((g, w), jnp.float32))
    else:
        nbuf = 4 if pipe else 1
        scratch += [
            pltpu.VMEM((nbuf, g, w), jnp.float32),
            pltpu.SemaphoreType.DMA((nbuf,)),
            pltpu.SemaphoreType.DMA((nbuf,)),
        ]
    return pl.kernel(
        body,
        out_type=jax.ShapeDtypeStruct((_SC_CORES * n, w), jnp.float32),
        mesh=mesh,
        scratch_types=scratch,
        compiler_params=pltpu.CompilerParams(use_tc_tiling_on_sc=False),
    )


def _seg_partials(y, src, dstls, n):
    w = y.shape[1]
    e = src.shape[0]
    g, _ = _seg_config(e, n, w, len(dstls))
    zeros = jnp.zeros((n // len(dstls) // _SC_SUBCORES, w), jnp.float32)
    kfn = _make_seg_kernel(n, e, w, len(dstls), False)
    out = kfn(y, src.reshape(e // g, g), *[d.reshape(e // g, g) for d in dstls],
              zeros)
    if w <= 128:
        return out.reshape(_SC_CORES, n * w // 128, 128)
    return out.reshape(_SC_CORES, n, w)


def _deg_partials(dstls, e, n):
    g, _ = _seg_config(e, n, 16, len(dstls))
    ones = jnp.ones((g, 16), jnp.float32)
    zeros = jnp.zeros((n // len(dstls) // _SC_SUBCORES, 16), jnp.float32)
    kfn = _make_seg_kernel(n, e, 16, len(dstls), True)
    out = kfn(ones, *[d.reshape(e // g, g) for d in dstls], zeros)
    return out.reshape(_SC_CORES, n // 8, 128)


_PNC = 8


@functools.lru_cache(maxsize=None)
def _make_partition_kernel(n, e):
    half = n // 2
    e_tile = e // _SC_TILES
    g = 128
    capg = ((e_tile // g + _PNC) // _PNC + 1) * _PNC
    cap = capg * g
    cbuf = 2048
    nch = e_tile // cbuf
    pad_unit = _PNC * g
    mesh = plsc.VectorSubcoreMesh(core_axis_name="c", subcore_axis_name="s")

    def body(src_hbm, dst_hbm, psrc_hbm, pdst_hbm, cnt_hbm,
             in_src, in_dst, lo_src, lo_dst, hi_src, hi_dst, cnt_v):
        c = lax.axis_index("c")
        s = lax.axis_index("s")
        tid = c * _SC_SUBCORES + s
        ebase = tid * e_tile

        def chunk(ch, offs):
            base = ebase + ch * cbuf
            pltpu.sync_copy(src_hbm.at[pl.ds(base, cbuf)], in_src)
            pltpu.sync_copy(dst_hbm.at[pl.ds(base, cbuf)], in_dst)

            def vec(v, offs2):
                off_lo, off_hi = offs2
                sv = in_src[pl.ds(v * 16, 16)]
                dv = in_dst[pl.ds(v * 16, 16)]
                mlo = dv < half
                nlo = jnp.max(plsc.all_reduce_population_count(mlo))
                dk, sp = plsc.sort_key_val(dv, sv)
                lo_src[pl.ds(off_lo, 16)] = sp
                lo_dst[pl.ds(off_lo, 16)] = dk
                dk2, sp2 = plsc.sort_key_val(dv, sv, descending=True)
                hi_src[pl.ds(off_hi, 16)] = sp2
                hi_dst[pl.ds(off_hi, 16)] = dk2 - half
                return off_lo + nlo, off_hi + (16 - nlo)

            return lax.fori_loop(0, cbuf // 16, vec, offs)

        off_lo, off_hi = lax.fori_loop(0, nch, chunk, (0, 0))

        zeros16 = jnp.zeros((16,), jnp.int32)
        garb16 = jnp.full((16,), half, jnp.int32)
        for p, (off, sbuf, dbuf) in enumerate(
                [(off_lo, lo_src, lo_dst), (off_hi, hi_src, hi_dst)]):
            npad = (pad_unit - off % pad_unit) % pad_unit

            def padv(k, _):
                sbuf[pl.ds(off + k * 16, 16)] = zeros16
                dbuf[pl.ds(off + k * 16, 16)] = garb16
                return 0

            lax.fori_loop(0, (npad + 15) // 16, padv, 0)
            total = off + npad
            pltpu.sync_copy(sbuf.at[pl.ds(0, cap)],
                            psrc_hbm.at[pl.ds((p * _SC_TILES + tid) * cap,
                                              cap)])
            pltpu.sync_copy(dbuf.at[pl.ds(0, cap)],
                            pdst_hbm.at[pl.ds((p * _SC_TILES + tid) * cap,
                                              cap)])
            cnt_v[...] = jnp.full((16,), total // pad_unit, jnp.int32)
            pltpu.sync_copy(cnt_v, cnt_hbm.at[p * _SC_TILES + tid])

    buf = cap + 32
    return pl.kernel(
        body,
        out_type=(
            jax.ShapeDtypeStruct((2 * _SC_TILES * cap,), jnp.int32),
            jax.ShapeDtypeStruct((2 * _SC_TILES * cap,), jnp.int32),
            jax.ShapeDtypeStruct((2 * _SC_TILES, 16), jnp.int32),
        ),
        mesh=mesh,
        scratch_types=[
            pltpu.VMEM((cbuf,), jnp.int32),
            pltpu.VMEM((cbuf,), jnp.int32),
            pltpu.VMEM((buf,), jnp.int32),
            pltpu.VMEM((buf,), jnp.int32),
            pltpu.VMEM((buf,), jnp.int32),
            pltpu.VMEM((buf,), jnp.int32),
            pltpu.VMEM((16,), jnp.int32),
        ],
        compiler_params=pltpu.CompilerParams(use_tc_tiling_on_sc=False,
                                             needs_layout_passes=False),
    )


def _partition_l0(src, dst, n):
    e = src.shape[0]
    e_tile = e // _SC_TILES
    capg = ((e_tile // 128 + _PNC) // _PNC + 1) * _PNC
    kfn = _make_partition_kernel(n, e)
    psrc, pdst, cnt = kfn(src, dst)
    return (psrc.reshape(2 * _SC_TILES * capg, 128),
            pdst.reshape(2 * _SC_TILES * capg, 128), cnt, capg)


@functools.lru_cache(maxsize=None)
def _make_seg_part_kernel(n, w, capg):
    half = n // 2
    half_pad = half + 128
    stripe = half // _SC_SUBCORES
    g = 128
    nc = _PNC
    mesh = plsc.VectorSubcoreMesh(core_axis_name="c", subcore_axis_name="s")

    def body(y_hbm, psrc_hbm, pdst_hbm, cnt_hbm, zeros_hbm, out_hbm,
             accum, srcv, dstv, cntv, rows, gsem, ssem):
        c = lax.axis_index("c")
        s = lax.axis_index("s")
        tid = c * _SC_SUBCORES + s
        for p in range(2):
            pltpu.sync_copy(zeros_hbm, accum.at[pl.ds(s * stripe, stripe)])
            pltpu.sync_copy(cnt_hbm.at[p * _SC_TILES + tid], cntv)
            plsc.subcore_barrier()
            nch = jnp.max(cntv[...])
            rowbase = (p * _SC_TILES + tid) * capg

            def chunkbody(ch, carry):
                gbase = rowbase + ch * nc
                pltpu.sync_copy(pdst_hbm.at[pl.ds(gbase, nc)], dstv)
                pltpu.sync_copy(psrc_hbm.at[pl.ds(gbase, nc)], srcv)
                for b in range(2):
                    pltpu.async_copy(y_hbm.at[srcv.at[b]], rows.at[b],
                                     gsem.at[b])

                def quadbody(i, cc):
                    for b in range(4):
                        j = 4 * i + b
                        bf = (b + 2) % 4

                        @pl.when(j < nc)
                        def _():
                            pltpu.make_async_copy(
                                y_hbm.at[srcv.at[0]], rows.at[b],
                                gsem.at[b]).wait()
                            pltpu.async_copy(rows.at[b], accum.at[dstv.at[j]],
                                             ssem.at[b], add=True)

                            @pl.when(j >= 2)
                            def _():
                                pltpu.make_async_copy(
                                    rows.at[bf], accum.at[dstv.at[0]],
                                    ssem.at[bf]).wait()

                            @pl.when(j + 2 < nc)
                            def _():
                                pltpu.async_copy(y_hbm.at[srcv.at[j + 2]],
                                                 rows.at[bf], gsem.at[bf])
                    return cc

                lax.fori_loop(0, (nc + 3) // 4, quadbody, 0)
                for jt in range(max(0, nc - 2), nc):
                    b = jt % 4
                    pltpu.make_async_copy(rows.at[b], accum.at[dstv.at[0]],
                                          ssem.at[b]).wait()
                return carry

            lax.fori_loop(0, nch, chunkbody, 0)
            plsc.subcore_barrier()
            pltpu.sync_copy(
                accum.at[pl.ds(s * stripe, stripe)],
                out_hbm.at[pl.ds(c * n + p * half + s * stripe, stripe)])
            if p == 0:
                plsc.subcore_barrier()

    return pl.kernel(
        body,
        out_type=jax.ShapeDtypeStruct((_SC_CORES * n, w), jnp.float32),
        mesh=mesh,
        scratch_types=[
            pltpu.VMEM_SHARED((half_pad, w), jnp.float32),
            pltpu.VMEM((nc, g), jnp.int32),
            pltpu.VMEM((nc, g), jnp.int32),
            pltpu.VMEM((16,), jnp.int32),
            pltpu.VMEM((4, g, w), jnp.float32),
            pltpu.SemaphoreType.DMA((4,)),
            pltpu.SemaphoreType.DMA((4,)),
        ],
        compiler_params=pltpu.CompilerParams(use_tc_tiling_on_sc=False,
                                             needs_layout_passes=False),
    )


def _seg_partials_part(y, part, n):
    psrc, pdst, cnt, capg = part
    w = y.shape[1]
    zeros = jnp.zeros((n // 2 // _SC_SUBCORES, w), jnp.float32)
    kfn = _make_seg_part_kernel(n, w, capg)
    out = kfn(y, psrc, pdst, cnt, zeros)
    return out.reshape(_SC_CORES, n * w // 128, 128)


def _dst_split_body(d_ref, lo_ref, hi_ref, *, half):
    d = d_ref[...]
    lo_ref[...] = jnp.where(d < half, d, half)
    hi_ref[...] = jnp.where(d >= half, d - half, half)


def _dst_split(dst, half):
    e = dst.shape[0]
    rows = e // 128
    x = dst.reshape(rows, 128)
    br = 512
    body = functools.partial(_dst_split_body, half=half)
    lo, hi = pl.pallas_call(
        body,
        grid=(rows // br,),
        in_specs=[pl.BlockSpec((br, 128), lambda i: (i, 0))],
        out_specs=[pl.BlockSpec((br, 128), lambda i: (i, 0))] * 2,
        out_shape=[jax.ShapeDtypeStruct((rows, 128), jnp.int32)] * 2,
    )(x)
    return lo.reshape(e), hi.reshape(e)


_SPMEM_BUDGET = 7 * 1024 * 1024


def _num_passes(n, w):
    p = 1
    while (n // p + 128) * w * 4 > _SPMEM_BUDGET:
        p *= 2
    return p


def _sage(x, src, dstls_fn, dp, params, name, n, relu=True, part=None):
    ws = params[name + "_Ws"]
    wn = params[name + "_Wn"]
    b = params[name + "_b"]
    ci, co = ws.shape
    w = min(ci, co)
    npass = _num_passes(n, w)

    def agg(y):
        if part is not None and npass == 2:
            return _seg_partials_part(y, part, n)
        return _seg_partials(y, src, dstls_fn(npass), n)

    if co < ci:
        y = _mm(x, wn)
        p = agg(y)
        return _combine(x, ws, None, p, dp, b, relu, w)
    p = agg(x)
    return _combine(x, ws, wn, p, dp, b, relu, w)


def kernel(in_feat, exteraVar1, params, edge_index1, edge_index2, edge_index3,
           edge_index4, edge_index5):
    del exteraVar1
    edges = [edge_index1, edge_index2, edge_index3, edge_index4, edge_index5]
    srcs = [e[0] for e in edges]
    dsts = [e[1] for e in edges]
    split_cache = {}

    def dstls_fn(lvl):
        def get(n_passes):
            if n_passes == 1:
                return [dsts[lvl]]
            key = (lvl, n_passes)
            if key not in split_cache:
                assert n_passes == 2
                split_cache[key] = list(_dst_split(dsts[lvl], _N[lvl] // 2))
            return split_cache[key]
        return get

    fns = [dstls_fn(l) for l in range(5)]
    dps = [_deg_partials(fns[l](_num_passes(_N[l], 16)), dsts[l].shape[0], _N[l])
           for l in range(5)]

    def sage(x, lvl, name, relu=True):
        return _sage(x, srcs[lvl], fns[lvl], dps[lvl], params, name, _N[lvl],
                     relu, part=None)

    h1 = sage(in_feat, 0, "conv1")
    h22 = sage(h1, 0, "conv2")
    h2 = _pool(h22, RES)
    h3 = sage(h2, 1, "conv3")
    h33 = sage(h3, 1, "conv33")
    h3p = _pool(h33, RES // 2)
    h4 = sage(h3p, 2, "conv4")
    h44 = sage(h4, 2, "conv44")
    h4p = _pool(h44, RES // 4)
    h5 = sage(h4p, 3, "conv5")
    h55 = sage(h5, 3, "conv55")
    h5p = _pool(h55, RES // 8)
    h6 = sage(h5p, 4, "conv6")
    h6 = sage(h6, 4, "conv66")
    h6 = sage(h6, 4, "conv7")
    h6 = _up(h6, RES // 16, params["up1_W"], params["up1_b"])
    h6 = jnp.concatenate([h6, h55], axis=1)
    h6 = sage(h6, 3, "conv7")
    h6 = sage(h6, 3, "conv77")
    h6 = sage(h6, 3, "conv8")
    h6 = _up(h6, RES // 8, params["up2_W"], params["up2_b"])
    h6 = jnp.concatenate([h6, h44], axis=1)
    h6 = sage(h6, 2, "conv8")
    h6 = sage(h6, 2, "conv88")
    h6 = sage(h6, 2, "conv9")
    h6 = _up(h6, RES // 4, params["up3_W"], params["up3_b"])
    h6 = jnp.concatenate([h6, h33], axis=1)
    h6 = sage(h6, 1, "conv9")
    h6 = sage(h6, 1, "conv99")
    h6 = sage(h6, 1, "conv10")
    h6 = _up(h6, RES // 2, params["up4_W"], params["up4_b"])
    h6 = jnp.concatenate([h6, h22], axis=1)
    h6 = sage(h6, 0, "conv10")
    h6 = sage(h6, 0, "conv101")
    return sage(h6, 0, "conv11", relu=False)


# speedup vs baseline: 1.1814x; 1.1200x over previous
import functools

import jax
import jax.numpy as jnp
from jax import lax
from jax.experimental import pallas as pl
from jax.experimental.pallas import tpu as pltpu
from jax.experimental.pallas import tpu_sc as plsc

RES = 128
P = 2
_N = [6 * (RES // (P ** k)) ** 2 for k in range(5)]

_SC_CORES = 2
_SC_SUBCORES = 16
_SC_TILES = _SC_CORES * _SC_SUBCORES


def _mm_body(x_ref, w_ref, b_ref, o_ref, *, relu):
    acc = jnp.dot(x_ref[...], w_ref[...], preferred_element_type=jnp.float32)
    if b_ref is not None:
        acc = acc + b_ref[...]
    if relu:
        acc = jnp.maximum(acc, 0.0)
    o_ref[...] = acc


def _mm(x, w, b=None, relu=False, bn=2048):
    n, ci = x.shape
    co = w.shape[1]
    bn = min(bn, n)
    grid = (n // bn,)
    in_specs = [
        pl.BlockSpec((bn, ci), lambda i: (i, 0)),
        pl.BlockSpec((ci, co), lambda i: (0, 0)),
    ]
    args = [x, w]
    if b is not None:
        in_specs.append(pl.BlockSpec((1, co), lambda i: (0, 0)))
        args.append(b.reshape(1, co))
    body = functools.partial(_mm_body, relu=relu)
    if b is None:
        body = lambda x_ref, w_ref, o_ref: _mm_body(x_ref, w_ref, None, o_ref, relu=relu)
    return pl.pallas_call(
        body,
        grid=grid,
        in_specs=in_specs,
        out_specs=pl.BlockSpec((bn, co), lambda i: (i, 0)),
        out_shape=jax.ShapeDtypeStruct((n, co), jnp.float32),
    )(*args)


def _mm2_body(a_ref, b_ref, wa_ref, wb_ref, o_ref):
    acc = jnp.dot(a_ref[...], wa_ref[...], preferred_element_type=jnp.float32)
    acc += jnp.dot(b_ref[...], wb_ref[...], preferred_element_type=jnp.float32)
    o_ref[...] = acc


def _mm2(a, b, wa, wb, bn=2048):
    n, ca = a.shape
    cb = b.shape[1]
    co = wa.shape[1]
    bn = min(bn, n)
    return pl.pallas_call(
        _mm2_body,
        grid=(n // bn,),
        in_specs=[
            pl.BlockSpec((bn, ca), lambda i: (i, 0)),
            pl.BlockSpec((bn, cb), lambda i: (i, 0)),
            pl.BlockSpec((ca, co), lambda i: (0, 0)),
            pl.BlockSpec((cb, co), lambda i: (0, 0)),
        ],
        out_specs=pl.BlockSpec((bn, co), lambda i: (i, 0)),
        out_shape=jax.ShapeDtypeStruct((n, co), jnp.float32),
    )(a, b, wa, wb)


def _combine_body(*refs, relu, nx, has_wn):
    x_refs = refs[:nx]
    ws_refs = refs[nx:2 * nx]
    i = 2 * nx
    wn_ref = refs[i] if has_wn else None
    i += 1 if has_wn else 0
    p_ref, dp_ref, b_ref, o_ref = refs[i:i + 4]
    deg = dp_ref[0, :, 0] + dp_ref[1, :, 0]
    invd = 1.0 / jnp.maximum(deg, 1.0)
    mean = (p_ref[0] + p_ref[1]) * invd[:, None]
    acc = jnp.dot(x_refs[0][...], ws_refs[0][...],
                  preferred_element_type=jnp.float32)
    for xr, wr in zip(x_refs[1:], ws_refs[1:]):
        acc += jnp.dot(xr[...], wr[...], preferred_element_type=jnp.float32)
    if wn_ref is not None:
        acc += jnp.dot(mean, wn_ref[...], preferred_element_type=jnp.float32)
    else:
        acc += mean
    acc += b_ref[...]
    if relu:
        acc = jnp.maximum(acc, 0.0)
    o_ref[...] = acc


def _combine(xs, wss, wn, p, dp, b, relu, bn=2048):
    n = xs[0].shape[0]
    co = wss[0].shape[1]
    w = p.shape[2]
    bn = min(bn, n)
    in_specs, args = [], []
    for x in xs:
        in_specs.append(pl.BlockSpec((bn, x.shape[1]), lambda i: (i, 0)))
        args.append(x)
    for ws in wss:
        in_specs.append(pl.BlockSpec(ws.shape, lambda i: (0, 0)))
        args.append(ws)
    if wn is not None:
        in_specs.append(pl.BlockSpec(wn.shape, lambda i: (0, 0)))
        args.append(wn)
    in_specs += [
        pl.BlockSpec((2, bn, w), lambda i: (0, i, 0)),
        pl.BlockSpec((2, bn, 16), lambda i: (0, i, 0)),
        pl.BlockSpec((1, co), lambda i: (0, 0)),
    ]
    args += [p, dp, b.reshape(1, co)]
    body = functools.partial(_combine_body, relu=relu, nx=len(xs),
                             has_wn=wn is not None)
    return pl.pallas_call(
        body,
        grid=(n // bn,),
        in_specs=in_specs,
        out_specs=pl.BlockSpec((bn, co), lambda i: (i, 0)),
        out_shape=jax.ShapeDtypeStruct((n, co), jnp.float32),
    )(*args)


def _pool_body(x_ref, o_ref):
    o_ref[...] = jnp.mean(x_ref[...], axis=(1, 3))


def _pool(h, res):
    c = h.shape[1]
    r2 = res // 2
    m = 6 * r2
    x = h.reshape(m, 2, r2, 2, c)
    g = 8 if m % 8 == 0 else 1
    out = pl.pallas_call(
        _pool_body,
        grid=(m // g,),
        in_specs=[pl.BlockSpec((g, 2, r2, 2, c), lambda i: (i, 0, 0, 0, 0))],
        out_specs=pl.BlockSpec((g, r2, c), lambda i: (i, 0, 0)),
        out_shape=jax.ShapeDtypeStruct((m, r2, c), jnp.float32),
    )(x)
    return out.reshape(m * r2, c)


def _up(h, res, w, b):
    c, d = w.shape[0], w.shape[1]
    wr = w.transpose(0, 2, 3, 1).reshape(c, 4 * d)
    b4 = jnp.tile(b, 4)
    p = _mm(h, wr, b4)
    p = p.reshape(6, res, res, 2, 2, d).transpose(0, 1, 3, 2, 4, 5)
    return p.reshape(6 * 4 * res * res, d)


def _seg_group_size(e_tile):
    for g in range(min(128, e_tile), 0, -8):
        if e_tile % g == 0:
            return g
    raise ValueError(e_tile)


def _seg_config(e, n, w, n_passes):
    e_tile = e // _SC_TILES
    accum_bytes = (n // n_passes + 128) * w * 4
    g = _seg_group_size(e_tile)
    while True:
        ng = e_tile // g
        nc = max(d for d in range(1, min(ng, 32) + 1) if ng % d == 0)
        tile_bytes = 2 * nc * g * 4 + 4 * g * w * 4
        if accum_bytes + _SC_SUBCORES * tile_bytes <= int(7.6 * 1024 * 1024):
            return g, nc
        ng2 = g
        for g2 in range(g - 8, 0, -8):
            if e_tile % g2 == 0:
                ng2 = g2
                break
        if ng2 == g:
            return g, nc
        g = ng2


@functools.lru_cache(maxsize=None)
def _make_seg_kernel(n, e, w, n_passes, ones_mode):
    half = n // n_passes
    half_pad = half + 128
    zstripe = half // _SC_SUBCORES
    stripe = half // _SC_SUBCORES
    e_tile = e // _SC_TILES
    g, nc = _seg_config(e, n, w, n_passes)
    ng = e_tile // g
    n_chunks = ng // nc
    pipe = nc >= 2 and not ones_mode
    mesh = plsc.VectorSubcoreMesh(core_axis_name="c", subcore_axis_name="s")

    def body(*refs):
        if ones_mode:
            ones_hbm, *dstls, zeros_hbm, out_hbm, accum, dstv, rows = refs
            src_hbm = srcv = gsem = ssem = None
        else:
            (y_hbm, src_hbm, *dstls, zeros_hbm, out_hbm,
             accum, srcv, dstv, rows, gsem, ssem) = refs
        c = lax.axis_index("c")
        s = lax.axis_index("s")
        tid = c * _SC_SUBCORES + s
        if ones_mode:
            pltpu.sync_copy(ones_hbm, rows)
        for p in range(n_passes):
            pltpu.sync_copy(zeros_hbm, accum.at[pl.ds(s * zstripe, zstripe)])
            plsc.subcore_barrier()
            dstl = dstls[p]

            def chunkbody(ch, carry):
                gbase = tid * ng + ch * nc
                pltpu.sync_copy(dstl.at[pl.ds(gbase, nc)], dstv)
                if ones_mode:
                    def grp(j, cc):
                        pltpu.sync_copy(rows, accum.at[dstv.at[j]], add=True)
                        return cc
                    lax.fori_loop(0, nc, grp, 0)
                elif not pipe:
                    pltpu.sync_copy(src_hbm.at[pl.ds(gbase, nc)], srcv)

                    def grp(j, cc):
                        pltpu.async_copy(y_hbm.at[srcv.at[j]], rows.at[0],
                                         gsem.at[0]).wait()
                        pltpu.sync_copy(rows.at[0], accum.at[dstv.at[j]],
                                        add=True)
                        return cc
                    lax.fori_loop(0, nc, grp, 0)
                else:
                    pltpu.sync_copy(src_hbm.at[pl.ds(gbase, nc)], srcv)
                    for b in range(min(2, nc)):
                        pltpu.async_copy(y_hbm.at[srcv.at[b]], rows.at[b],
                                         gsem.at[b])

                    def quadbody(i, cc):
                        for b in range(4):
                            j = 4 * i + b

                            bf = (b + 2) % 4

                            @pl.when(j < nc)
                            def _():
                                pltpu.make_async_copy(
                                    y_hbm.at[srcv.at[0]], rows.at[b],
                                    gsem.at[b]).wait()
                                pltpu.async_copy(rows.at[b],
                                                 accum.at[dstv.at[j]],
                                                 ssem.at[b], add=True)

                                @pl.when(j >= 2)
                                def _():
                                    pltpu.make_async_copy(
                                        rows.at[bf], accum.at[dstv.at[0]],
                                        ssem.at[bf]).wait()

                                @pl.when(j + 2 < nc)
                                def _():
                                    pltpu.async_copy(y_hbm.at[srcv.at[j + 2]],
                                                     rows.at[bf], gsem.at[bf])
                        return cc

                    lax.fori_loop(0, (nc + 3) // 4, quadbody, 0)
                    for jt in range(max(0, nc - 2), nc):
                        b = jt % 4
                        pltpu.make_async_copy(rows.at[b], accum.at[dstv.at[0]],
                                              ssem.at[b]).wait()
                return carry

            lax.fori_loop(0, n_chunks, chunkbody, 0)
            plsc.subcore_barrier()
            pltpu.sync_copy(
                accum.at[pl.ds(s * stripe, stripe)],
                out_hbm.at[pl.ds(c * n + p * half + s * stripe, stripe)])
            if p + 1 < n_passes:
                plsc.subcore_barrier()

    scratch = [pltpu.VMEM_SHARED((half_pad, w), jnp.float32)]
    if not ones_mode:
        scratch.append(pltpu.VMEM((nc, g), jnp.int32))
    scratch.append(pltpu.VMEM((nc, g), jnp.int32))
    if ones_mode:
        scratch.append(pltpu.VMEM((g, w), jnp.float32))
    else:
        nbuf = 4 if pipe else 1
        scratch += [
            pltpu.VMEM((nbuf, g, w), jnp.float32),
            pltpu.SemaphoreType.DMA((nbuf,)),
            pltpu.SemaphoreType.DMA((nbuf,)),
        ]
    return pl.kernel(
        body,
        out_type=jax.ShapeDtypeStruct((_SC_CORES * n, w), jnp.float32),
        mesh=mesh,
        scratch_types=scratch,
        compiler_params=pltpu.CompilerParams(use_tc_tiling_on_sc=False),
    )


def _seg_partials(y, src, dstls, n):
    w = y.shape[1]
    e = src.shape[0]
    g, _ = _seg_config(e, n, w, len(dstls))
    zeros = jnp.zeros((n // len(dstls) // _SC_SUBCORES, w), jnp.float32)
    kfn = _make_seg_kernel(n, e, w, len(dstls), False)
    out = kfn(y, src.reshape(e // g, g), *[d.reshape(e // g, g) for d in dstls],
              zeros)
    return out.reshape(_SC_CORES, n, w)


def _deg_partials(dstls, e, n):
    g, _ = _seg_config(e, n, 16, len(dstls))
    ones = jnp.ones((g, 16), jnp.float32)
    zeros = jnp.zeros((n // len(dstls) // _SC_SUBCORES, 16), jnp.float32)
    kfn = _make_seg_kernel(n, e, 16, len(dstls), True)
    out = kfn(ones, *[d.reshape(e // g, g) for d in dstls], zeros)
    return out.reshape(_SC_CORES, n, 16)


_PNC = 8


@functools.lru_cache(maxsize=None)
def _make_partition_kernel(n, e):
    half = n // 2
    e_tile = e // _SC_TILES
    g = 128
    capg = ((e_tile // g + _PNC) // _PNC + 1) * _PNC
    cap = capg * g
    cbuf = 2048
    nch = e_tile // cbuf
    pad_unit = _PNC * g
    mesh = plsc.VectorSubcoreMesh(core_axis_name="c", subcore_axis_name="s")

    def body(src_hbm, dst_hbm, psrc_hbm, pdst_hbm, cnt_hbm,
             in_src, in_dst, lo_src, lo_dst, hi_src, hi_dst, cnt_v):
        c = lax.axis_index("c")
        s = lax.axis_index("s")
        tid = c * _SC_SUBCORES + s
        ebase = tid * e_tile

        def chunk(ch, offs):
            base = ebase + ch * cbuf
            pltpu.sync_copy(src_hbm.at[pl.ds(base, cbuf)], in_src)
            pltpu.sync_copy(dst_hbm.at[pl.ds(base, cbuf)], in_dst)

            def vec(v, offs2):
                off_lo, off_hi = offs2
                sv = in_src[pl.ds(v * 16, 16)]
                dv = in_dst[pl.ds(v * 16, 16)]
                mlo = dv < half
                nlo = jnp.max(plsc.all_reduce_population_count(mlo))
                dk, sp = plsc.sort_key_val(dv, sv)
                lo_src[pl.ds(off_lo, 16)] = sp
                lo_dst[pl.ds(off_lo, 16)] = dk
                dk2, sp2 = plsc.sort_key_val(dv, sv, descending=True)
                hi_src[pl.ds(off_hi, 16)] = sp2
                hi_dst[pl.ds(off_hi, 16)] = dk2 - half
                return off_lo + nlo, off_hi + (16 - nlo)

            return lax.fori_loop(0, cbuf // 16, vec, offs)

        off_lo, off_hi = lax.fori_loop(0, nch, chunk, (0, 0))

        zeros16 = jnp.zeros((16,), jnp.int32)
        garb16 = jnp.full((16,), half, jnp.int32)
        for p, (off, sbuf, dbuf) in enumerate(
                [(off_lo, lo_src, lo_dst), (off_hi, hi_src, hi_dst)]):
            npad = (pad_unit - off % pad_unit) % pad_unit

            def padv(k, _):
                sbuf[pl.ds(off + k * 16, 16)] = zeros16
                dbuf[pl.ds(off + k * 16, 16)] = garb16
                return 0

            lax.fori_loop(0, (npad + 15) // 16, padv, 0)
            total = off + npad
            pltpu.sync_copy(sbuf.at[pl.ds(0, cap)],
                            psrc_hbm.at[pl.ds((p * _SC_TILES + tid) * cap,
                                              cap)])
            pltpu.sync_copy(dbuf.at[pl.ds(0, cap)],
                            pdst_hbm.at[pl.ds((p * _SC_TILES + tid) * cap,
                                              cap)])
            cnt_v[...] = jnp.full((16,), total // pad_unit, jnp.int32)
            pltpu.sync_copy(cnt_v, cnt_hbm.at[p * _SC_TILES + tid])

    buf = cap + 32
    return pl.kernel(
        body,
        out_type=(
            jax.ShapeDtypeStruct((2 * _SC_TILES * cap,), jnp.int32),
            jax.ShapeDtypeStruct((2 * _SC_TILES * cap,), jnp.int32),
            jax.ShapeDtypeStruct((2 * _SC_TILES, 16), jnp.int32),
        ),
        mesh=mesh,
        scratch_types=[
            pltpu.VMEM((cbuf,), jnp.int32),
            pltpu.VMEM((cbuf,), jnp.int32),
            pltpu.VMEM((buf,), jnp.int32),
            pltpu.VMEM((buf,), jnp.int32),
            pltpu.VMEM((buf,), jnp.int32),
            pltpu.VMEM((buf,), jnp.int32),
            pltpu.VMEM((16,), jnp.int32),
        ],
        compiler_params=pltpu.CompilerParams(use_tc_tiling_on_sc=False,
                                             needs_layout_passes=False),
    )


def _partition_l0(src, dst, n):
    e = src.shape[0]
    e_tile = e // _SC_TILES
    capg = ((e_tile // 128 + _PNC) // _PNC + 1) * _PNC
    kfn = _make_partition_kernel(n, e)
    psrc, pdst, cnt = kfn(src, dst)
    return (psrc.reshape(2 * _SC_TILES * capg, 128),
            pdst.reshape(2 * _SC_TILES * capg, 128), cnt, capg)


@functools.lru_cache(maxsize=None)
def _make_seg_part_kernel(n, w, capg):
    half = n // 2
    half_pad = half + 128
    stripe = half // _SC_SUBCORES
    g = 128
    nc = _PNC
    mesh = plsc.VectorSubcoreMesh(core_axis_name="c", subcore_axis_name="s")

    def body(y_hbm, psrc_hbm, pdst_hbm, cnt_hbm, zeros_hbm, out_hbm,
             accum, srcv, dstv, cntv, rows, gsem, ssem):
        c = lax.axis_index("c")
        s = lax.axis_index("s")
        tid = c * _SC_SUBCORES + s
        for p in range(2):
            pltpu.sync_copy(zeros_hbm, accum.at[pl.ds(s * stripe, stripe)])
            pltpu.sync_copy(cnt_hbm.at[p * _SC_TILES + tid], cntv)
            plsc.subcore_barrier()
            nch = jnp.max(cntv[...])
            rowbase = (p * _SC_TILES + tid) * capg

            def chunkbody(ch, carry):
                gbase = rowbase + ch * nc
                pltpu.sync_copy(pdst_hbm.at[pl.ds(gbase, nc)], dstv)
                pltpu.sync_copy(psrc_hbm.at[pl.ds(gbase, nc)], srcv)
                for b in range(2):
                    pltpu.async_copy(y_hbm.at[srcv.at[b]], rows.at[b],
                                     gsem.at[b])

                def quadbody(i, cc):
                    for b in range(4):
                        j = 4 * i + b
                        bf = (b + 2) % 4

                        @pl.when(j < nc)
                        def _():
                            pltpu.make_async_copy(
                                y_hbm.at[srcv.at[0]], rows.at[b],
                                gsem.at[b]).wait()
                            pltpu.async_copy(rows.at[b], accum.at[dstv.at[j]],
                                             ssem.at[b], add=True)

                            @pl.when(j >= 2)
                            def _():
                                pltpu.make_async_copy(
                                    rows.at[bf], accum.at[dstv.at[0]],
                                    ssem.at[bf]).wait()

                            @pl.when(j + 2 < nc)
                            def _():
                                pltpu.async_copy(y_hbm.at[srcv.at[j + 2]],
                                                 rows.at[bf], gsem.at[bf])
                    return cc

                lax.fori_loop(0, (nc + 3) // 4, quadbody, 0)
                for jt in range(max(0, nc - 2), nc):
                    b = jt % 4
                    pltpu.make_async_copy(rows.at[b], accum.at[dstv.at[0]],
                                          ssem.at[b]).wait()
                return carry

            lax.fori_loop(0, nch, chunkbody, 0)
            plsc.subcore_barrier()
            pltpu.sync_copy(
                accum.at[pl.ds(s * stripe, stripe)],
                out_hbm.at[pl.ds(c * n + p * half + s * stripe, stripe)])
            if p == 0:
                plsc.subcore_barrier()

    return pl.kernel(
        body,
        out_type=jax.ShapeDtypeStruct((_SC_CORES * n, w), jnp.float32),
        mesh=mesh,
        scratch_types=[
            pltpu.VMEM_SHARED((half_pad, w), jnp.float32),
            pltpu.VMEM((nc, g), jnp.int32),
            pltpu.VMEM((nc, g), jnp.int32),
            pltpu.VMEM((16,), jnp.int32),
            pltpu.VMEM((4, g, w), jnp.float32),
            pltpu.SemaphoreType.DMA((4,)),
            pltpu.SemaphoreType.DMA((4,)),
        ],
        compiler_params=pltpu.CompilerParams(use_tc_tiling_on_sc=False,
                                             needs_layout_passes=False),
    )


def _seg_partials_part(y, part, n):
    psrc, pdst, cnt, capg = part
    w = y.shape[1]
    zeros = jnp.zeros((n // 2 // _SC_SUBCORES, w), jnp.float32)
    kfn = _make_seg_part_kernel(n, w, capg)
    out = kfn(y, psrc, pdst, cnt, zeros)
    return out.reshape(_SC_CORES, n, w)


def _dst_split_body(d_ref, lo_ref, hi_ref, *, half):
    d = d_ref[...]
    lo_ref[...] = jnp.where(d < half, d, half)
    hi_ref[...] = jnp.where(d >= half, d - half, half)


def _dst_split(dst, half):
    e = dst.shape[0]
    rows = e // 128
    x = dst.reshape(rows, 128)
    br = 512
    body = functools.partial(_dst_split_body, half=half)
    lo, hi = pl.pallas_call(
        body,
        grid=(rows // br,),
        in_specs=[pl.BlockSpec((br, 128), lambda i: (i, 0))],
        out_specs=[pl.BlockSpec((br, 128), lambda i: (i, 0))] * 2,
        out_shape=[jax.ShapeDtypeStruct((rows, 128), jnp.int32)] * 2,
    )(x)
    return lo.reshape(e), hi.reshape(e)


_SPMEM_BUDGET = 7 * 1024 * 1024


def _num_passes(n, w):
    p = 1
    while (n // p + 128) * w * 4 > _SPMEM_BUDGET:
        p *= 2
    return p


def _sage(xs, src, dstls_fn, dp, params, name, n, relu=True, part=None):
    ws = params[name + "_Ws"]
    wn = params[name + "_Wn"]
    b = params[name + "_b"]
    ci, co = ws.shape
    w = min(ci, co)
    npass = _num_passes(n, w)
    offs = [0]
    for x in xs:
        offs.append(offs[-1] + x.shape[1])
    assert offs[-1] == ci
    wss = [ws[offs[i]:offs[i + 1]] for i in range(len(xs))]
    wns = [wn[offs[i]:offs[i + 1]] for i in range(len(xs))]

    def agg(y):
        if part is not None and npass == 2:
            return _seg_partials_part(y, part, n)
        return _seg_partials(y, src, dstls_fn(npass), n)

    if co < ci:
        if len(xs) == 1:
            y = _mm(xs[0], wns[0])
        else:
            y = _mm2(xs[0], xs[1], wns[0], wns[1])
        p = agg(y)
        return _combine(xs, wss, None, p, dp, b, relu)
    assert len(xs) == 1
    p = agg(xs[0])
    return _combine(xs, wss, wn, p, dp, b, relu)


def kernel(in_feat, exteraVar1, params, edge_index1, edge_index2, edge_index3,
           edge_index4, edge_index5):
    del exteraVar1
    edges = [edge_index1, edge_index2, edge_index3, edge_index4, edge_index5]
    srcs = [e[0] for e in edges]
    dsts = [e[1] for e in edges]
    split_cache = {}

    def dstls_fn(lvl):
        def get(n_passes):
            if n_passes == 1:
                return [dsts[lvl]]
            key = (lvl, n_passes)
            if key not in split_cache:
                assert n_passes == 2
                split_cache[key] = list(_dst_split(dsts[lvl], _N[lvl] // 2))
            return split_cache[key]
        return get

    fns = [dstls_fn(l) for l in range(5)]
    dps = [_deg_partials(fns[l](_num_passes(_N[l], 16)), dsts[l].shape[0], _N[l])
           for l in range(5)]

    def sage(xs, lvl, name, relu=True):
        if not isinstance(xs, list):
            xs = [xs]
        return _sage(xs, srcs[lvl], fns[lvl], dps[lvl], params, name, _N[lvl],
                     relu, part=None)

    h1 = sage(in_feat, 0, "conv1")
    h22 = sage(h1, 0, "conv2")
    h2 = _pool(h22, RES)
    h3 = sage(h2, 1, "conv3")
    h33 = sage(h3, 1, "conv33")
    h3p = _pool(h33, RES // 2)
    h4 = sage(h3p, 2, "conv4")
    h44 = sage(h4, 2, "conv44")
    h4p = _pool(h44, RES // 4)
    h5 = sage(h4p, 3, "conv5")
    h55 = sage(h5, 3, "conv55")
    h5p = _pool(h55, RES // 8)
    h6 = sage(h5p, 4, "conv6")
    h6 = sage(h6, 4, "conv66")
    h6 = sage(h6, 4, "conv7")
    h6 = _up(h6, RES // 16, params["up1_W"], params["up1_b"])
    h6 = sage([h6, h55], 3, "conv7")
    h6 = sage(h6, 3, "conv77")
    h6 = sage(h6, 3, "conv8")
    h6 = _up(h6, RES // 8, params["up2_W"], params["up2_b"])
    h6 = sage([h6, h44], 2, "conv8")
    h6 = sage(h6, 2, "conv88")
    h6 = sage(h6, 2, "conv9")
    h6 = _up(h6, RES // 4, params["up3_W"], params["up3_b"])
    h6 = sage([h6, h33], 1, "conv9")
    h6 = sage(h6, 1, "conv99")
    h6 = sage(h6, 1, "conv10")
    h6 = _up(h6, RES // 2, params["up4_W"], params["up4_b"])
    h6 = sage([h6, h22], 0, "conv10")
    h6 = sage(h6, 0, "conv101")
    return sage(h6, 0, "conv11", relu=False)


# speedup vs baseline: 1.1817x; 1.0003x over previous
import functools

import jax
import jax.numpy as jnp
from jax import lax
from jax.experimental import pallas as pl
from jax.experimental.pallas import tpu as pltpu
from jax.experimental.pallas import tpu_sc as plsc

RES = 128
P = 2
_N = [6 * (RES // (P ** k)) ** 2 for k in range(5)]

_SC_CORES = 2
_SC_SUBCORES = 16
_SC_TILES = _SC_CORES * _SC_SUBCORES


def _mm_body(x_ref, w_ref, b_ref, o_ref, *, relu):
    acc = jnp.dot(x_ref[...], w_ref[...], preferred_element_type=jnp.float32)
    if b_ref is not None:
        acc = acc + b_ref[...]
    if relu:
        acc = jnp.maximum(acc, 0.0)
    o_ref[...] = acc


def _mm(x, w, b=None, relu=False, bn=2048):
    n, ci = x.shape
    co = w.shape[1]
    bn = min(bn, n)
    grid = (n // bn,)
    in_specs = [
        pl.BlockSpec((bn, ci), lambda i: (i, 0)),
        pl.BlockSpec((ci, co), lambda i: (0, 0)),
    ]
    args = [x, w]
    if b is not None:
        in_specs.append(pl.BlockSpec((1, co), lambda i: (0, 0)))
        args.append(b.reshape(1, co))
    body = functools.partial(_mm_body, relu=relu)
    if b is None:
        body = lambda x_ref, w_ref, o_ref: _mm_body(x_ref, w_ref, None, o_ref, relu=relu)
    return pl.pallas_call(
        body,
        grid=grid,
        in_specs=in_specs,
        out_specs=pl.BlockSpec((bn, co), lambda i: (i, 0)),
        out_shape=jax.ShapeDtypeStruct((n, co), jnp.float32),
    )(*args)


def _mm2_body(a_ref, b_ref, wa_ref, wb_ref, o_ref):
    acc = jnp.dot(a_ref[...], wa_ref[...], preferred_element_type=jnp.float32)
    acc += jnp.dot(b_ref[...], wb_ref[...], preferred_element_type=jnp.float32)
    o_ref[...] = acc


def _mm2(a, b, wa, wb, bn=2048):
    n, ca = a.shape
    cb = b.shape[1]
    co = wa.shape[1]
    bn = min(bn, n)
    return pl.pallas_call(
        _mm2_body,
        grid=(n // bn,),
        in_specs=[
            pl.BlockSpec((bn, ca), lambda i: (i, 0)),
            pl.BlockSpec((bn, cb), lambda i: (i, 0)),
            pl.BlockSpec((ca, co), lambda i: (0, 0)),
            pl.BlockSpec((cb, co), lambda i: (0, 0)),
        ],
        out_specs=pl.BlockSpec((bn, co), lambda i: (i, 0)),
        out_shape=jax.ShapeDtypeStruct((n, co), jnp.float32),
    )(a, b, wa, wb)


def _combine_body(*refs, relu, nx, has_wn):
    x_refs = refs[:nx]
    ws_refs = refs[nx:2 * nx]
    i = 2 * nx
    wn_ref = refs[i] if has_wn else None
    i += 1 if has_wn else 0
    p_ref, dp_ref, b_ref, o_ref = refs[i:i + 4]
    deg = dp_ref[0, :, 0] + dp_ref[1, :, 0]
    invd = 1.0 / jnp.maximum(deg, 1.0)
    mean = (p_ref[0] + p_ref[1]) * invd[:, None]
    acc = jnp.dot(x_refs[0][...], ws_refs[0][...],
                  preferred_element_type=jnp.float32)
    for xr, wr in zip(x_refs[1:], ws_refs[1:]):
        acc += jnp.dot(xr[...], wr[...], preferred_element_type=jnp.float32)
    if wn_ref is not None:
        acc += jnp.dot(mean, wn_ref[...], preferred_element_type=jnp.float32)
    else:
        acc += mean
    acc += b_ref[...]
    if relu:
        acc = jnp.maximum(acc, 0.0)
    o_ref[...] = acc


def _combine(xs, wss, wn, p, dp, b, relu, bn=2048):
    n = xs[0].shape[0]
    co = wss[0].shape[1]
    w = p.shape[2]
    bn = min(bn, n)
    in_specs, args = [], []
    for x in xs:
        in_specs.append(pl.BlockSpec((bn, x.shape[1]), lambda i: (i, 0)))
        args.append(x)
    for ws in wss:
        in_specs.append(pl.BlockSpec(ws.shape, lambda i: (0, 0)))
        args.append(ws)
    if wn is not None:
        in_specs.append(pl.BlockSpec(wn.shape, lambda i: (0, 0)))
        args.append(wn)
    in_specs += [
        pl.BlockSpec((2, bn, w), lambda i: (0, i, 0)),
        pl.BlockSpec((2, bn, 16), lambda i: (0, i, 0)),
        pl.BlockSpec((1, co), lambda i: (0, 0)),
    ]
    args += [p, dp, b.reshape(1, co)]
    body = functools.partial(_combine_body, relu=relu, nx=len(xs),
                             has_wn=wn is not None)
    return pl.pallas_call(
        body,
        grid=(n // bn,),
        in_specs=in_specs,
        out_specs=pl.BlockSpec((bn, co), lambda i: (i, 0)),
        out_shape=jax.ShapeDtypeStruct((n, co), jnp.float32),
    )(*args)


def _pool_body(x_ref, o_ref):
    o_ref[...] = jnp.mean(x_ref[...], axis=(1, 3))


def _pool(h, res):
    c = h.shape[1]
    r2 = res // 2
    m = 6 * r2
    x = h.reshape(m, 2, r2, 2, c)
    g = 8 if m % 8 == 0 else 1
    out = pl.pallas_call(
        _pool_body,
        grid=(m // g,),
        in_specs=[pl.BlockSpec((g, 2, r2, 2, c), lambda i: (i, 0, 0, 0, 0))],
        out_specs=pl.BlockSpec((g, r2, c), lambda i: (i, 0, 0)),
        out_shape=jax.ShapeDtypeStruct((m, r2, c), jnp.float32),
    )(x)
    return out.reshape(m * r2, c)


def _up(h, res, w, b):
    c, d = w.shape[0], w.shape[1]
    wr = w.transpose(0, 2, 3, 1).reshape(c, 4 * d)
    b4 = jnp.tile(b, 4)
    p = _mm(h, wr, b4)
    p = p.reshape(6, res, res, 2, 2, d).transpose(0, 1, 3, 2, 4, 5)
    return p.reshape(6 * 4 * res * res, d)


def _seg_group_size(e_tile):
    for g in range(min(128, e_tile), 0, -8):
        if e_tile % g == 0:
            return g
    raise ValueError(e_tile)


def _seg_config(e, n, w, n_passes):
    e_tile = e // _SC_TILES
    accum_bytes = (n // n_passes + 128) * w * 4
    g = _seg_group_size(e_tile)
    while True:
        ng = e_tile // g
        nc = max(d for d in range(1, min(ng, 32) + 1) if ng % d == 0)
        tile_bytes = 2 * nc * g * 4 + 4 * g * w * 4
        if accum_bytes + _SC_SUBCORES * tile_bytes <= int(7.6 * 1024 * 1024):
            return g, nc
        ng2 = g
        for g2 in range(g - 8, 0, -8):
            if e_tile % g2 == 0:
                ng2 = g2
                break
        if ng2 == g:
            return g, nc
        g = ng2


@functools.lru_cache(maxsize=None)
def _make_seg_kernel(n, e, w, n_passes, ones_mode):
    half = n // n_passes
    half_pad = half + 128
    zstripe = half // _SC_SUBCORES
    stripe = half // _SC_SUBCORES
    e_tile = e // _SC_TILES
    g, nc = _seg_config(e, n, w, n_passes)
    ng = e_tile // g
    n_chunks = ng // nc
    pipe = nc >= 2 and not ones_mode
    mesh = plsc.VectorSubcoreMesh(core_axis_name="c", subcore_axis_name="s")

    def body(*refs):
        if ones_mode:
            ones_hbm, *dstls, zeros_hbm, out_hbm, accum, dstv, rows, ssem = refs
            src_hbm = srcv = gsem = None
        else:
            (y_hbm, src_hbm, *dstls, zeros_hbm, out_hbm,
             accum, srcv, dstv, rows, gsem, ssem) = refs
        c = lax.axis_index("c")
        s = lax.axis_index("s")
        tid = c * _SC_SUBCORES + s
        if ones_mode:
            pltpu.sync_copy(ones_hbm, rows)
        for p in range(n_passes):
            pltpu.sync_copy(zeros_hbm, accum.at[pl.ds(s * zstripe, zstripe)])
            plsc.subcore_barrier()
            dstl = dstls[p]

            def chunkbody(ch, carry):
                gbase = tid * ng + ch * nc
                pltpu.sync_copy(dstl.at[pl.ds(gbase, nc)], dstv)
                if ones_mode:
                    def quad1(i, cc):
                        for b in range(4):
                            j = 4 * i + b

                            @pl.when(j < nc)
                            def _():
                                @pl.when(j >= 4)
                                def _():
                                    pltpu.make_async_copy(
                                        rows, accum.at[dstv.at[0]],
                                        ssem.at[b]).wait()
                                pltpu.async_copy(rows, accum.at[dstv.at[j]],
                                                 ssem.at[b], add=True)
                        return cc

                    lax.fori_loop(0, (nc + 3) // 4, quad1, 0)
                    for jt in range(max(0, nc - 4), nc):
                        pltpu.make_async_copy(rows, accum.at[dstv.at[0]],
                                              ssem.at[jt % 4]).wait()
                elif not pipe:
                    pltpu.sync_copy(src_hbm.at[pl.ds(gbase, nc)], srcv)

                    def grp(j, cc):
                        pltpu.async_copy(y_hbm.at[srcv.at[j]], rows.at[0],
                                         gsem.at[0]).wait()
                        pltpu.sync_copy(rows.at[0], accum.at[dstv.at[j]],
                                        add=True)
                        return cc
                    lax.fori_loop(0, nc, grp, 0)
                else:
                    pltpu.sync_copy(src_hbm.at[pl.ds(gbase, nc)], srcv)
                    for b in range(min(2, nc)):
                        pltpu.async_copy(y_hbm.at[srcv.at[b]], rows.at[b],
                                         gsem.at[b])

                    def quadbody(i, cc):
                        for b in range(4):
                            j = 4 * i + b

                            bf = (b + 2) % 4

                            @pl.when(j < nc)
                            def _():
                                pltpu.make_async_copy(
                                    y_hbm.at[srcv.at[0]], rows.at[b],
                                    gsem.at[b]).wait()
                                pltpu.async_copy(rows.at[b],
                                                 accum.at[dstv.at[j]],
                                                 ssem.at[b], add=True)

                                @pl.when(j >= 2)
                                def _():
                                    pltpu.make_async_copy(
                                        rows.at[bf], accum.at[dstv.at[0]],
                                        ssem.at[bf]).wait()

                                @pl.when(j + 2 < nc)
                                def _():
                                    pltpu.async_copy(y_hbm.at[srcv.at[j + 2]],
                                                     rows.at[bf], gsem.at[bf])
                        return cc

                    lax.fori_loop(0, (nc + 3) // 4, quadbody, 0)
                    for jt in range(max(0, nc - 2), nc):
                        b = jt % 4
                        pltpu.make_async_copy(rows.at[b], accum.at[dstv.at[0]],
                                              ssem.at[b]).wait()
                return carry

            lax.fori_loop(0, n_chunks, chunkbody, 0)
            plsc.subcore_barrier()
            pltpu.sync_copy(
                accum.at[pl.ds(s * stripe, stripe)],
                out_hbm.at[pl.ds(c * n + p * half + s * stripe, stripe)])
            if p + 1 < n_passes:
                plsc.subcore_barrier()

    scratch = [pltpu.VMEM_SHARED((half_pad, w), jnp.float32)]
    if not ones_mode:
        scratch.append(pltpu.VMEM((nc, g), jnp.int32))
    scratch.append(pltpu.VMEM((nc, g), jnp.int32))
    if ones_mode:
        scratch += [
            pltpu.VMEM((g, w), jnp.float32),
            pltpu.SemaphoreType.DMA((4,)),
        ]
    else:
        nbuf = 4 if pipe else 1
        scratch += [
            pltpu.VMEM((nbuf, g, w), jnp.float32),
            pltpu.SemaphoreType.DMA((nbuf,)),
            pltpu.SemaphoreType.DMA((nbuf,)),
        ]
    return pl.kernel(
        body,
        out_type=jax.ShapeDtypeStruct((_SC_CORES * n, w), jnp.float32),
        mesh=mesh,
        scratch_types=scratch,
        compiler_params=pltpu.CompilerParams(use_tc_tiling_on_sc=False),
    )


def _seg_partials(y, src, dstls, n):
    w = y.shape[1]
    e = src.shape[0]
    g, _ = _seg_config(e, n, w, len(dstls))
    zeros = jnp.zeros((n // len(dstls) // _SC_SUBCORES, w), jnp.float32)
    kfn = _make_seg_kernel(n, e, w, len(dstls), False)
    out = kfn(y, src.reshape(e // g, g), *[d.reshape(e // g, g) for d in dstls],
              zeros)
    return out.reshape(_SC_CORES, n, w)


def _deg_partials(dstls, e, n):
    g, _ = _seg_config(e, n, 16, len(dstls))
    ones = jnp.ones((g, 16), jnp.float32)
    zeros = jnp.zeros((n // len(dstls) // _SC_SUBCORES, 16), jnp.float32)
    kfn = _make_seg_kernel(n, e, 16, len(dstls), True)
    out = kfn(ones, *[d.reshape(e // g, g) for d in dstls], zeros)
    return out.reshape(_SC_CORES, n, 16)


_PNC = 8


@functools.lru_cache(maxsize=None)
def _make_partition_kernel(n, e):
    half = n // 2
    e_tile = e // _SC_TILES
    g = 128
    capg = ((e_tile // g + _PNC) // _PNC + 1) * _PNC
    cap = capg * g
    cbuf = 2048
    nch = e_tile // cbuf
    pad_unit = _PNC * g
    mesh = plsc.VectorSubcoreMesh(core_axis_name="c", subcore_axis_name="s")

    def body(src_hbm, dst_hbm, psrc_hbm, pdst_hbm, cnt_hbm,
             in_src, in_dst, lo_src, lo_dst, hi_src, hi_dst, cnt_v):
        c = lax.axis_index("c")
        s = lax.axis_index("s")
        tid = c * _SC_SUBCORES + s
        ebase = tid * e_tile

        def chunk(ch, offs):
            base = ebase + ch * cbuf
            pltpu.sync_copy(src_hbm.at[pl.ds(base, cbuf)], in_src)
            pltpu.sync_copy(dst_hbm.at[pl.ds(base, cbuf)], in_dst)

            def vec(v, offs2):
                off_lo, off_hi = offs2
                sv = in_src[pl.ds(v * 16, 16)]
                dv = in_dst[pl.ds(v * 16, 16)]
                mlo = dv < half
                nlo = jnp.max(plsc.all_reduce_population_count(mlo))
                dk, sp = plsc.sort_key_val(dv, sv)
                lo_src[pl.ds(off_lo, 16)] = sp
                lo_dst[pl.ds(off_lo, 16)] = dk
                dk2, sp2 = plsc.sort_key_val(dv, sv, descending=True)
                hi_src[pl.ds(off_hi, 16)] = sp2
                hi_dst[pl.ds(off_hi, 16)] = dk2 - half
                return off_lo + nlo, off_hi + (16 - nlo)

            return lax.fori_loop(0, cbuf // 16, vec, offs)

        off_lo, off_hi = lax.fori_loop(0, nch, chunk, (0, 0))

        zeros16 = jnp.zeros((16,), jnp.int32)
        garb16 = jnp.full((16,), half, jnp.int32)
        for p, (off, sbuf, dbuf) in enumerate(
                [(off_lo, lo_src, lo_dst), (off_hi, hi_src, hi_dst)]):
            npad = (pad_unit - off % pad_unit) % pad_unit

            def padv(k, _):
                sbuf[pl.ds(off + k * 16, 16)] = zeros16
                dbuf[pl.ds(off + k * 16, 16)] = garb16
                return 0

            lax.fori_loop(0, (npad + 15) // 16, padv, 0)
            total = off + npad
            pltpu.sync_copy(sbuf.at[pl.ds(0, cap)],
                            psrc_hbm.at[pl.ds((p * _SC_TILES + tid) * cap,
                                              cap)])
            pltpu.sync_copy(dbuf.at[pl.ds(0, cap)],
                            pdst_hbm.at[pl.ds((p * _SC_TILES + tid) * cap,
                                              cap)])
            cnt_v[...] = jnp.full((16,), total // pad_unit, jnp.int32)
            pltpu.sync_copy(cnt_v, cnt_hbm.at[p * _SC_TILES + tid])

    buf = cap + 32
    return pl.kernel(
        body,
        out_type=(
            jax.ShapeDtypeStruct((2 * _SC_TILES * cap,), jnp.int32),
            jax.ShapeDtypeStruct((2 * _SC_TILES * cap,), jnp.int32),
            jax.ShapeDtypeStruct((2 * _SC_TILES, 16), jnp.int32),
        ),
        mesh=mesh,
        scratch_types=[
            pltpu.VMEM((cbuf,), jnp.int32),
            pltpu.VMEM((cbuf,), jnp.int32),
            pltpu.VMEM((buf,), jnp.int32),
            pltpu.VMEM((buf,), jnp.int32),
            pltpu.VMEM((buf,), jnp.int32),
            pltpu.VMEM((buf,), jnp.int32),
            pltpu.VMEM((16,), jnp.int32),
        ],
        compiler_params=pltpu.CompilerParams(use_tc_tiling_on_sc=False,
                                             needs_layout_passes=False),
    )


def _partition_l0(src, dst, n):
    e = src.shape[0]
    e_tile = e // _SC_TILES
    capg = ((e_tile // 128 + _PNC) // _PNC + 1) * _PNC
    kfn = _make_partition_kernel(n, e)
    psrc, pdst, cnt = kfn(src, dst)
    return (psrc.reshape(2 * _SC_TILES * capg, 128),
            pdst.reshape(2 * _SC_TILES * capg, 128), cnt, capg)


@functools.lru_cache(maxsize=None)
def _make_seg_part_kernel(n, w, capg):
    half = n // 2
    half_pad = half + 128
    stripe = half // _SC_SUBCORES
    g = 128
    nc = _PNC
    mesh = plsc.VectorSubcoreMesh(core_axis_name="c", subcore_axis_name="s")

    def body(y_hbm, psrc_hbm, pdst_hbm, cnt_hbm, zeros_hbm, out_hbm,
             accum, srcv, dstv, cntv, rows, gsem, ssem):
        c = lax.axis_index("c")
        s = lax.axis_index("s")
        tid = c * _SC_SUBCORES + s
        for p in range(2):
            pltpu.sync_copy(zeros_hbm, accum.at[pl.ds(s * stripe, stripe)])
            pltpu.sync_copy(cnt_hbm.at[p * _SC_TILES + tid], cntv)
            plsc.subcore_barrier()
            nch = jnp.max(cntv[...])
            rowbase = (p * _SC_TILES + tid) * capg

            def chunkbody(ch, carry):
                gbase = rowbase + ch * nc
                pltpu.sync_copy(pdst_hbm.at[pl.ds(gbase, nc)], dstv)
                pltpu.sync_copy(psrc_hbm.at[pl.ds(gbase, nc)], srcv)
                for b in range(2):
                    pltpu.async_copy(y_hbm.at[srcv.at[b]], rows.at[b],
                                     gsem.at[b])

                def quadbody(i, cc):
                    for b in range(4):
                        j = 4 * i + b
                        bf = (b + 2) % 4

                        @pl.when(j < nc)
                        def _():
                            pltpu.make_async_copy(
                                y_hbm.at[srcv.at[0]], rows.at[b],
                                gsem.at[b]).wait()
                            pltpu.async_copy(rows.at[b], accum.at[dstv.at[j]],
                                             ssem.at[b], add=True)

                            @pl.when(j >= 2)
                            def _():
                                pltpu.make_async_copy(
                                    rows.at[bf], accum.at[dstv.at[0]],
                                    ssem.at[bf]).wait()

                            @pl.when(j + 2 < nc)
                            def _():
                                pltpu.async_copy(y_hbm.at[srcv.at[j + 2]],
                                                 rows.at[bf], gsem.at[bf])
                    return cc

                lax.fori_loop(0, (nc + 3) // 4, quadbody, 0)
                for jt in range(max(0, nc - 2), nc):
                    b = jt % 4
                    pltpu.make_async_copy(rows.at[b], accum.at[dstv.at[0]],
                                          ssem.at[b]).wait()
                return carry

            lax.fori_loop(0, nch, chunkbody, 0)
            plsc.subcore_barrier()
            pltpu.sync_copy(
                accum.at[pl.ds(s * stripe, stripe)],
                out_hbm.at[pl.ds(c * n + p * half + s * stripe, stripe)])
            if p == 0:
                plsc.subcore_barrier()

    return pl.kernel(
        body,
        out_type=jax.ShapeDtypeStruct((_SC_CORES * n, w), jnp.float32),
        mesh=mesh,
        scratch_types=[
            pltpu.VMEM_SHARED((half_pad, w), jnp.float32),
            pltpu.VMEM((nc, g), jnp.int32),
            pltpu.VMEM((nc, g), jnp.int32),
            pltpu.VMEM((16,), jnp.int32),
            pltpu.VMEM((4, g, w), jnp.float32),
            pltpu.SemaphoreType.DMA((4,)),
            pltpu.SemaphoreType.DMA((4,)),
        ],
        compiler_params=pltpu.CompilerParams(use_tc_tiling_on_sc=False,
                                             needs_layout_passes=False),
    )


def _seg_partials_part(y, part, n):
    psrc, pdst, cnt, capg = part
    w = y.shape[1]
    zeros = jnp.zeros((n // 2 // _SC_SUBCORES, w), jnp.float32)
    kfn = _make_seg_part_kernel(n, w, capg)
    out = kfn(y, psrc, pdst, cnt, zeros)
    return out.reshape(_SC_CORES, n, w)


def _dst_split_body(d_ref, lo_ref, hi_ref, *, half):
    d = d_ref[...]
    lo_ref[...] = jnp.where(d < half, d, half)
    hi_ref[...] = jnp.where(d >= half, d - half, half)


def _dst_split(dst, half):
    e = dst.shape[0]
    rows = e // 128
    x = dst.reshape(rows, 128)
    br = 512
    body = functools.partial(_dst_split_body, half=half)
    lo, hi = pl.pallas_call(
        body,
        grid=(rows // br,),
        in_specs=[pl.BlockSpec((br, 128), lambda i: (i, 0))],
        out_specs=[pl.BlockSpec((br, 128), lambda i: (i, 0))] * 2,
        out_shape=[jax.ShapeDtypeStruct((rows, 128), jnp.int32)] * 2,
    )(x)
    return lo.reshape(e), hi.reshape(e)


_SPMEM_BUDGET = 7 * 1024 * 1024


def _num_passes(n, w):
    p = 1
    while (n // p + 128) * w * 4 > _SPMEM_BUDGET:
        p *= 2
    return p


def _sage(xs, src, dstls_fn, dp, params, name, n, relu=True, part=None):
    ws = params[name + "_Ws"]
    wn = params[name + "_Wn"]
    b = params[name + "_b"]
    ci, co = ws.shape
    w = min(ci, co)
    npass = _num_passes(n, w)
    offs = [0]
    for x in xs:
        offs.append(offs[-1] + x.shape[1])
    assert offs[-1] == ci
    wss = [ws[offs[i]:offs[i + 1]] for i in range(len(xs))]
    wns = [wn[offs[i]:offs[i + 1]] for i in range(len(xs))]

    def agg(y):
        if part is not None and npass == 2:
            return _seg_partials_part(y, part, n)
        return _seg_partials(y, src, dstls_fn(npass), n)

    if co < ci:
        if len(xs) == 1:
            y = _mm(xs[0], wns[0])
        else:
            y = _mm2(xs[0], xs[1], wns[0], wns[1])
        p = agg(y)
        return _combine(xs, wss, None, p, dp, b, relu)
    assert len(xs) == 1
    p = agg(xs[0])
    return _combine(xs, wss, wn, p, dp, b, relu)


def kernel(in_feat, exteraVar1, params, edge_index1, edge_index2, edge_index3,
           edge_index4, edge_index5):
    del exteraVar1
    edges = [edge_index1, edge_index2, edge_index3, edge_index4, edge_index5]
    srcs = [e[0] for e in edges]
    dsts = [e[1] for e in edges]
    split_cache = {}

    def dstls_fn(lvl):
        def get(n_passes):
            if n_passes == 1:
                return [dsts[lvl]]
            key = (lvl, n_passes)
            if key not in split_cache:
                assert n_passes == 2
                split_cache[key] = list(_dst_split(dsts[lvl], _N[lvl] // 2))
            return split_cache[key]
        return get

    fns = [dstls_fn(l) for l in range(5)]
    dps = [_deg_partials(fns[l](_num_passes(_N[l], 16)), dsts[l].shape[0], _N[l])
           for l in range(5)]

    def sage(xs, lvl, name, relu=True):
        if not isinstance(xs, list):
            xs = [xs]
        return _sage(xs, srcs[lvl], fns[lvl], dps[lvl], params, name, _N[lvl],
                     relu, part=None)

    h1 = sage(in_feat, 0, "conv1")
    h22 = sage(h1, 0, "conv2")
    h2 = _pool(h22, RES)
    h3 = sage(h2, 1, "conv3")
    h33 = sage(h3, 1, "conv33")
    h3p = _pool(h33, RES // 2)
    h4 = sage(h3p, 2, "conv4")
    h44 = sage(h4, 2, "conv44")
    h4p = _pool(h44, RES // 4)
    h5 = sage(h4p, 3, "conv5")
    h55 = sage(h5, 3, "conv55")
    h5p = _pool(h55, RES // 8)
    h6 = sage(h5p, 4, "conv6")
    h6 = sage(h6, 4, "conv66")
    h6 = sage(h6, 4, "conv7")
    h6 = _up(h6, RES // 16, params["up1_W"], params["up1_b"])
    h6 = sage([h6, h55], 3, "conv7")
    h6 = sage(h6, 3, "conv77")
    h6 = sage(h6, 3, "conv8")
    h6 = _up(h6, RES // 8, params["up2_W"], params["up2_b"])
    h6 = sage([h6, h44], 2, "conv8")
    h6 = sage(h6, 2, "conv88")
    h6 = sage(h6, 2, "conv9")
    h6 = _up(h6, RES // 4, params["up3_W"], params["up3_b"])
    h6 = sage([h6, h33], 1, "conv9")
    h6 = sage(h6, 1, "conv99")
    h6 = sage(h6, 1, "conv10")
    h6 = _up(h6, RES // 2, params["up4_W"], params["up4_b"])
    h6 = sage([h6, h22], 0, "conv10")
    h6 = sage(h6, 0, "conv101")
    return sage(h6, 0, "conv11", relu=False)


# speedup vs baseline: 1.5557x; 1.3165x over previous
import functools

import jax
import jax.numpy as jnp
from jax import lax
from jax.experimental import pallas as pl
from jax.experimental.pallas import tpu as pltpu
from jax.experimental.pallas import tpu_sc as plsc

RES = 128
P = 2
_N = [6 * (RES // (P ** k)) ** 2 for k in range(5)]

_SC_CORES = 2
_SC_SUBCORES = 16
_SC_TILES = _SC_CORES * _SC_SUBCORES


def _mm_body(x_ref, w_ref, b_ref, o_ref, *, relu):
    acc = jnp.dot(x_ref[...], w_ref[...], preferred_element_type=jnp.float32)
    if b_ref is not None:
        acc = acc + b_ref[...]
    if relu:
        acc = jnp.maximum(acc, 0.0)
    o_ref[...] = acc


def _mm(x, w, b=None, relu=False, bn=2048):
    n, ci = x.shape
    co = w.shape[1]
    bn = min(bn, n)
    grid = (n // bn,)
    in_specs = [
        pl.BlockSpec((bn, ci), lambda i: (i, 0)),
        pl.BlockSpec((ci, co), lambda i: (0, 0)),
    ]
    args = [x, w]
    if b is not None:
        in_specs.append(pl.BlockSpec((1, co), lambda i: (0, 0)))
        args.append(b.reshape(1, co))
    body = functools.partial(_mm_body, relu=relu)
    if b is None:
        body = lambda x_ref, w_ref, o_ref: _mm_body(x_ref, w_ref, None, o_ref, relu=relu)
    return pl.pallas_call(
        body,
        grid=grid,
        in_specs=in_specs,
        out_specs=pl.BlockSpec((bn, co), lambda i: (i, 0)),
        out_shape=jax.ShapeDtypeStruct((n, co), jnp.float32),
    )(*args)


def _mm2_body(a_ref, b_ref, wa_ref, wb_ref, o_ref):
    acc = jnp.dot(a_ref[...], wa_ref[...], preferred_element_type=jnp.float32)
    acc += jnp.dot(b_ref[...], wb_ref[...], preferred_element_type=jnp.float32)
    o_ref[...] = acc


def _mm2(a, b, wa, wb, bn=2048):
    n, ca = a.shape
    cb = b.shape[1]
    co = wa.shape[1]
    bn = min(bn, n)
    return pl.pallas_call(
        _mm2_body,
        grid=(n // bn,),
        in_specs=[
            pl.BlockSpec((bn, ca), lambda i: (i, 0)),
            pl.BlockSpec((bn, cb), lambda i: (i, 0)),
            pl.BlockSpec((ca, co), lambda i: (0, 0)),
            pl.BlockSpec((cb, co), lambda i: (0, 0)),
        ],
        out_specs=pl.BlockSpec((bn, co), lambda i: (i, 0)),
        out_shape=jax.ShapeDtypeStruct((n, co), jnp.float32),
    )(a, b, wa, wb)


def _combine_body(*refs, relu, nx, has_wn, colsplit):
    x_refs = refs[:nx]
    ws_refs = refs[nx:2 * nx]
    i = 2 * nx
    wn_ref = refs[i] if has_wn else None
    i += 1 if has_wn else 0
    p_ref, dp_ref, b_ref, o_ref = refs[i:i + 4]
    deg = dp_ref[0, :, 0] + dp_ref[1, :, 0]
    invd = 1.0 / jnp.maximum(deg, 1.0)
    if colsplit:
        mean_raw = jnp.concatenate([p_ref[0], p_ref[1]], axis=-1)
    else:
        mean_raw = p_ref[0] + p_ref[1]
    mean = mean_raw * invd[:, None]
    acc = jnp.dot(x_refs[0][...], ws_refs[0][...],
                  preferred_element_type=jnp.float32)
    for xr, wr in zip(x_refs[1:], ws_refs[1:]):
        acc += jnp.dot(xr[...], wr[...], preferred_element_type=jnp.float32)
    if wn_ref is not None:
        acc += jnp.dot(mean, wn_ref[...], preferred_element_type=jnp.float32)
    else:
        acc += mean
    acc += b_ref[...]
    if relu:
        acc = jnp.maximum(acc, 0.0)
    o_ref[...] = acc


def _combine(xs, wss, wn, p, dp, b, relu, colsplit=False, bn=2048):
    n = xs[0].shape[0]
    co = wss[0].shape[1]
    w = p.shape[2]
    bn = min(bn, n)
    in_specs, args = [], []
    for x in xs:
        in_specs.append(pl.BlockSpec((bn, x.shape[1]), lambda i: (i, 0)))
        args.append(x)
    for ws in wss:
        in_specs.append(pl.BlockSpec(ws.shape, lambda i: (0, 0)))
        args.append(ws)
    if wn is not None:
        in_specs.append(pl.BlockSpec(wn.shape, lambda i: (0, 0)))
        args.append(wn)
    in_specs += [
        pl.BlockSpec((2, bn, w), lambda i: (0, i, 0)),
        pl.BlockSpec((2, bn, 16), lambda i: (0, i, 0)),
        pl.BlockSpec((1, co), lambda i: (0, 0)),
    ]
    args += [p, dp, b.reshape(1, co)]
    body = functools.partial(_combine_body, relu=relu, nx=len(xs),
                             has_wn=wn is not None, colsplit=colsplit)
    return pl.pallas_call(
        body,
        grid=(n // bn,),
        in_specs=in_specs,
        out_specs=pl.BlockSpec((bn, co), lambda i: (i, 0)),
        out_shape=jax.ShapeDtypeStruct((n, co), jnp.float32),
    )(*args)


def _pool_body(x_ref, o_ref):
    o_ref[...] = jnp.mean(x_ref[...], axis=(1, 3))


def _pool(h, res):
    c = h.shape[1]
    r2 = res // 2
    m = 6 * r2
    x = h.reshape(m, 2, r2, 2, c)
    g = 8 if m % 8 == 0 else 1
    out = pl.pallas_call(
        _pool_body,
        grid=(m // g,),
        in_specs=[pl.BlockSpec((g, 2, r2, 2, c), lambda i: (i, 0, 0, 0, 0))],
        out_specs=pl.BlockSpec((g, r2, c), lambda i: (i, 0, 0)),
        out_shape=jax.ShapeDtypeStruct((m, r2, c), jnp.float32),
    )(x)
    return out.reshape(m * r2, c)


def _up(h, res, w, b):
    c, d = w.shape[0], w.shape[1]
    wr = w.transpose(0, 2, 3, 1).reshape(c, 4 * d)
    b4 = jnp.tile(b, 4)
    p = _mm(h, wr, b4)
    p = p.reshape(6, res, res, 2, 2, d).transpose(0, 1, 3, 2, 4, 5)
    return p.reshape(6 * 4 * res * res, d)


def _seg_group_size(e_tile):
    for g in range(min(128, e_tile), 0, -8):
        if e_tile % g == 0:
            return g
    raise ValueError(e_tile)


def _seg_config(e, n, w, n_passes):
    e_tile = e // _SC_TILES
    accum_bytes = (n // n_passes + 128) * w * 4
    g = _seg_group_size(e_tile)
    while True:
        ng = e_tile // g
        nc = max(d for d in range(1, min(ng, 32) + 1) if ng % d == 0)
        tile_bytes = 2 * nc * g * 4 + 4 * g * w * 4
        if accum_bytes + _SC_SUBCORES * tile_bytes <= int(7.6 * 1024 * 1024):
            return g, nc
        ng2 = g
        for g2 in range(g - 8, 0, -8):
            if e_tile % g2 == 0:
                ng2 = g2
                break
        if ng2 == g:
            return g, nc
        g = ng2


@functools.lru_cache(maxsize=None)
def _make_seg_colsplit_kernel(n, e, w):
    wh = w // 2
    n_pad = n + 128
    zstripe = n // _SC_SUBCORES
    stripe = n // _SC_SUBCORES
    e_tile = e // _SC_SUBCORES
    g = 128
    ng = e_tile // g
    nc = max(d for d in range(1, min(ng, 32) + 1) if ng % d == 0)
    mesh = plsc.VectorSubcoreMesh(core_axis_name="c", subcore_axis_name="s")

    def body(y_hbm, src_hbm, dst_hbm, zeros_hbm, out_hbm,
             accum, srcv, dstv, rows, gsem, ssem):
        c = lax.axis_index("c")
        s = lax.axis_index("s")
        pltpu.sync_copy(zeros_hbm, accum.at[pl.ds(s * zstripe, zstripe)])
        plsc.subcore_barrier()
        gbase0 = s * ng

        def chunkbody(ch, carry):
            gbase = gbase0 + ch * nc
            pltpu.sync_copy(dst_hbm.at[pl.ds(gbase, nc)], dstv)
            pltpu.sync_copy(src_hbm.at[pl.ds(gbase * g, nc * g)], srcv)

            def xform(v, cc):
                sl = srcv[pl.ds(v * 16, 16)]
                srcv[pl.ds(v * 16, 16)] = sl * 2 + c
                return cc

            lax.fori_loop(0, nc * g // 16, xform, 0)
            for b in range(min(2, nc)):
                pltpu.async_copy(y_hbm.at[srcv.at[pl.ds(b * g, g)]],
                                 rows.at[b], gsem.at[b])

            def quadbody(i, cc):
                for b in range(4):
                    j = 4 * i + b
                    bf = (b + 2) % 4

                    @pl.when(j < nc)
                    def _():
                        pltpu.make_async_copy(
                            y_hbm.at[srcv.at[pl.ds(0, g)]], rows.at[b],
                            gsem.at[b]).wait()
                        pltpu.async_copy(rows.at[b], accum.at[dstv.at[j]],
                                         ssem.at[b], add=True)

                        @pl.when(j >= 2)
                        def _():
                            pltpu.make_async_copy(
                                rows.at[bf], accum.at[dstv.at[0]],
                                ssem.at[bf]).wait()

                        @pl.when(j + 2 < nc)
                        def _():
                            pltpu.async_copy(
                                y_hbm.at[srcv.at[pl.ds((j + 2) * g, g)]],
                                rows.at[bf], gsem.at[bf])
                return cc

            lax.fori_loop(0, (nc + 3) // 4, quadbody, 0)
            for jt in range(max(0, nc - 2), nc):
                b = jt % 4
                pltpu.make_async_copy(rows.at[b], accum.at[dstv.at[0]],
                                      ssem.at[b]).wait()
            return carry

        lax.fori_loop(0, ng // nc, chunkbody, 0)
        plsc.subcore_barrier()
        pltpu.sync_copy(
            accum.at[pl.ds(s * stripe, stripe)],
            out_hbm.at[pl.ds(c * n + s * stripe, stripe)])

    return pl.kernel(
        body,
        out_type=jax.ShapeDtypeStruct((_SC_CORES * n, wh), jnp.float32),
        mesh=mesh,
        scratch_types=[
            pltpu.VMEM_SHARED((n_pad, wh), jnp.float32),
            pltpu.VMEM((nc * g,), jnp.int32),
            pltpu.VMEM((nc, g), jnp.int32),
            pltpu.VMEM((4, g, wh), jnp.float32),
            pltpu.SemaphoreType.DMA((4,)),
            pltpu.SemaphoreType.DMA((4,)),
        ],
        compiler_params=pltpu.CompilerParams(use_tc_tiling_on_sc=False),
    )


def _seg_partials_colsplit(y, src, dst, n):
    w = y.shape[1]
    e = src.shape[0]
    zeros = jnp.zeros((n // _SC_SUBCORES, w // 2), jnp.float32)
    kfn = _make_seg_colsplit_kernel(n, e, w)
    out = kfn(y.reshape(2 * y.shape[0], w // 2), src,
              dst.reshape(e // 128, 128), zeros)
    return out.reshape(_SC_CORES, n, w // 2)


@functools.lru_cache(maxsize=None)
def _make_seg_kernel(n, e, w, n_passes, ones_mode):
    half = n // n_passes
    half_pad = half + 128
    zstripe = half // _SC_SUBCORES
    stripe = half // _SC_SUBCORES
    e_tile = e // _SC_TILES
    g, nc = _seg_config(e, n, w, n_passes)
    ng = e_tile // g
    n_chunks = ng // nc
    pipe = nc >= 2 and not ones_mode
    mesh = plsc.VectorSubcoreMesh(core_axis_name="c", subcore_axis_name="s")

    def body(*refs):
        if ones_mode:
            ones_hbm, *dstls, zeros_hbm, out_hbm, accum, dstv, rows, ssem = refs
            src_hbm = srcv = gsem = None
        else:
            (y_hbm, src_hbm, *dstls, zeros_hbm, out_hbm,
             accum, srcv, dstv, rows, gsem, ssem) = refs
        c = lax.axis_index("c")
        s = lax.axis_index("s")
        tid = c * _SC_SUBCORES + s
        if ones_mode:
            pltpu.sync_copy(ones_hbm, rows)
        for p in range(n_passes):
            pltpu.sync_copy(zeros_hbm, accum.at[pl.ds(s * zstripe, zstripe)])
            plsc.subcore_barrier()
            dstl = dstls[p]

            def chunkbody(ch, carry):
                gbase = tid * ng + ch * nc
                pltpu.sync_copy(dstl.at[pl.ds(gbase, nc)], dstv)
                if ones_mode:
                    def quad1(i, cc):
                        for b in range(4):
                            j = 4 * i + b

                            @pl.when(j < nc)
                            def _():
                                @pl.when(j >= 4)
                                def _():
                                    pltpu.make_async_copy(
                                        rows, accum.at[dstv.at[0]],
                                        ssem.at[b]).wait()
                                pltpu.async_copy(rows, accum.at[dstv.at[j]],
                                                 ssem.at[b], add=True)
                        return cc

                    lax.fori_loop(0, (nc + 3) // 4, quad1, 0)
                    for jt in range(max(0, nc - 4), nc):
                        pltpu.make_async_copy(rows, accum.at[dstv.at[0]],
                                              ssem.at[jt % 4]).wait()
                elif not pipe:
                    pltpu.sync_copy(src_hbm.at[pl.ds(gbase, nc)], srcv)

                    def grp(j, cc):
                        pltpu.async_copy(y_hbm.at[srcv.at[j]], rows.at[0],
                                         gsem.at[0]).wait()
                        pltpu.sync_copy(rows.at[0], accum.at[dstv.at[j]],
                                        add=True)
                        return cc
                    lax.fori_loop(0, nc, grp, 0)
                else:
                    pltpu.sync_copy(src_hbm.at[pl.ds(gbase, nc)], srcv)
                    for b in range(min(2, nc)):
                        pltpu.async_copy(y_hbm.at[srcv.at[b]], rows.at[b],
                                         gsem.at[b])

                    def quadbody(i, cc):
                        for b in range(4):
                            j = 4 * i + b

                            bf = (b + 2) % 4

                            @pl.when(j < nc)
                            def _():
                                pltpu.make_async_copy(
                                    y_hbm.at[srcv.at[0]], rows.at[b],
                                    gsem.at[b]).wait()
                                pltpu.async_copy(rows.at[b],
                                                 accum.at[dstv.at[j]],
                                                 ssem.at[b], add=True)

                                @pl.when(j >= 2)
                                def _():
                                    pltpu.make_async_copy(
                                        rows.at[bf], accum.at[dstv.at[0]],
                                        ssem.at[bf]).wait()

                                @pl.when(j + 2 < nc)
                                def _():
                                    pltpu.async_copy(y_hbm.at[srcv.at[j + 2]],
                                                     rows.at[bf], gsem.at[bf])
                        return cc

                    lax.fori_loop(0, (nc + 3) // 4, quadbody, 0)
                    for jt in range(max(0, nc - 2), nc):
                        b = jt % 4
                        pltpu.make_async_copy(rows.at[b], accum.at[dstv.at[0]],
                                              ssem.at[b]).wait()
                return carry

            lax.fori_loop(0, n_chunks, chunkbody, 0)
            plsc.subcore_barrier()
            pltpu.sync_copy(
                accum.at[pl.ds(s * stripe, stripe)],
                out_hbm.at[pl.ds(c * n + p * half + s * stripe, stripe)])
            if p + 1 < n_passes:
                plsc.subcore_barrier()

    scratch = [pltpu.VMEM_SHARED((half_pad, w), jnp.float32)]
    if not ones_mode:
        scratch.append(pltpu.VMEM((nc, g), jnp.int32))
    scratch.append(pltpu.VMEM((nc, g), jnp.int32))
    if ones_mode:
        scratch += [
            pltpu.VMEM((g, w), jnp.float32),
            pltpu.SemaphoreType.DMA((4,)),
        ]
    else:
        nbuf = 4 if pipe else 1
        scratch += [
            pltpu.VMEM((nbuf, g, w), jnp.float32),
            pltpu.SemaphoreType.DMA((nbuf,)),
            pltpu.SemaphoreType.DMA((nbuf,)),
        ]
    return pl.kernel(
        body,
        out_type=jax.ShapeDtypeStruct((_SC_CORES * n, w), jnp.float32),
        mesh=mesh,
        scratch_types=scratch,
        compiler_params=pltpu.CompilerParams(use_tc_tiling_on_sc=False),
    )


def _seg_partials(y, src, dstls, n):
    w = y.shape[1]
    e = src.shape[0]
    g, _ = _seg_config(e, n, w, len(dstls))
    zeros = jnp.zeros((n // len(dstls) // _SC_SUBCORES, w), jnp.float32)
    kfn = _make_seg_kernel(n, e, w, len(dstls), False)
    out = kfn(y, src.reshape(e // g, g), *[d.reshape(e // g, g) for d in dstls],
              zeros)
    return out.reshape(_SC_CORES, n, w)


def _deg_partials(dstls, e, n):
    g, _ = _seg_config(e, n, 16, len(dstls))
    ones = jnp.ones((g, 16), jnp.float32)
    zeros = jnp.zeros((n // len(dstls) // _SC_SUBCORES, 16), jnp.float32)
    kfn = _make_seg_kernel(n, e, 16, len(dstls), True)
    out = kfn(ones, *[d.reshape(e // g, g) for d in dstls], zeros)
    return out.reshape(_SC_CORES, n, 16)


_PNC = 8


@functools.lru_cache(maxsize=None)
def _make_partition_kernel(n, e):
    half = n // 2
    e_tile = e // _SC_TILES
    g = 128
    capg = ((e_tile // g + _PNC) // _PNC + 1) * _PNC
    cap = capg * g
    cbuf = 2048
    nch = e_tile // cbuf
    pad_unit = _PNC * g
    mesh = plsc.VectorSubcoreMesh(core_axis_name="c", subcore_axis_name="s")

    def body(src_hbm, dst_hbm, psrc_hbm, pdst_hbm, cnt_hbm,
             in_src, in_dst, lo_src, lo_dst, hi_src, hi_dst, cnt_v):
        c = lax.axis_index("c")
        s = lax.axis_index("s")
        tid = c * _SC_SUBCORES + s
        ebase = tid * e_tile

        def chunk(ch, offs):
            base = ebase + ch * cbuf
            pltpu.sync_copy(src_hbm.at[pl.ds(base, cbuf)], in_src)
            pltpu.sync_copy(dst_hbm.at[pl.ds(base, cbuf)], in_dst)

            def vec(v, offs2):
                off_lo, off_hi = offs2
                sv = in_src[pl.ds(v * 16, 16)]
                dv = in_dst[pl.ds(v * 16, 16)]
                mlo = dv < half
                nlo = jnp.max(plsc.all_reduce_population_count(mlo))
                dk, sp = plsc.sort_key_val(dv, sv)
                lo_src[pl.ds(off_lo, 16)] = sp
                lo_dst[pl.ds(off_lo, 16)] = dk
                dk2, sp2 = plsc.sort_key_val(dv, sv, descending=True)
                hi_src[pl.ds(off_hi, 16)] = sp2
                hi_dst[pl.ds(off_hi, 16)] = dk2 - half
                return off_lo + nlo, off_hi + (16 - nlo)

            return lax.fori_loop(0, cbuf // 16, vec, offs)

        off_lo, off_hi = lax.fori_loop(0, nch, chunk, (0, 0))

        zeros16 = jnp.zeros((16,), jnp.int32)
        garb16 = jnp.full((16,), half, jnp.int32)
        for p, (off, sbuf, dbuf) in enumerate(
                [(off_lo, lo_src, lo_dst), (off_hi, hi_src, hi_dst)]):
            npad = (pad_unit - off % pad_unit) % pad_unit

            def padv(k, _):
                sbuf[pl.ds(off + k * 16, 16)] = zeros16
                dbuf[pl.ds(off + k * 16, 16)] = garb16
                return 0

            lax.fori_loop(0, (npad + 15) // 16, padv, 0)
            total = off + npad
            pltpu.sync_copy(sbuf.at[pl.ds(0, cap)],
                            psrc_hbm.at[pl.ds((p * _SC_TILES + tid) * cap,
                                              cap)])
            pltpu.sync_copy(dbuf.at[pl.ds(0, cap)],
                            pdst_hbm.at[pl.ds((p * _SC_TILES + tid) * cap,
                                              cap)])
            cnt_v[...] = jnp.full((16,), total // pad_unit, jnp.int32)
            pltpu.sync_copy(cnt_v, cnt_hbm.at[p * _SC_TILES + tid])

    buf = cap + 32
    return pl.kernel(
        body,
        out_type=(
            jax.ShapeDtypeStruct((2 * _SC_TILES * cap,), jnp.int32),
            jax.ShapeDtypeStruct((2 * _SC_TILES * cap,), jnp.int32),
            jax.ShapeDtypeStruct((2 * _SC_TILES, 16), jnp.int32),
        ),
        mesh=mesh,
        scratch_types=[
            pltpu.VMEM((cbuf,), jnp.int32),
            pltpu.VMEM((cbuf,), jnp.int32),
            pltpu.VMEM((buf,), jnp.int32),
            pltpu.VMEM((buf,), jnp.int32),
            pltpu.VMEM((buf,), jnp.int32),
            pltpu.VMEM((buf,), jnp.int32),
            pltpu.VMEM((16,), jnp.int32),
        ],
        compiler_params=pltpu.CompilerParams(use_tc_tiling_on_sc=False,
                                             needs_layout_passes=False),
    )


def _partition_l0(src, dst, n):
    e = src.shape[0]
    e_tile = e // _SC_TILES
    capg = ((e_tile // 128 + _PNC) // _PNC + 1) * _PNC
    kfn = _make_partition_kernel(n, e)
    psrc, pdst, cnt = kfn(src, dst)
    return (psrc.reshape(2 * _SC_TILES * capg, 128),
            pdst.reshape(2 * _SC_TILES * capg, 128), cnt, capg)


@functools.lru_cache(maxsize=None)
def _make_seg_part_kernel(n, w, capg):
    half = n // 2
    half_pad = half + 128
    stripe = half // _SC_SUBCORES
    g = 128
    nc = _PNC
    mesh = plsc.VectorSubcoreMesh(core_axis_name="c", subcore_axis_name="s")

    def body(y_hbm, psrc_hbm, pdst_hbm, cnt_hbm, zeros_hbm, out_hbm,
             accum, srcv, dstv, cntv, rows, gsem, ssem):
        c = lax.axis_index("c")
        s = lax.axis_index("s")
        tid = c * _SC_SUBCORES + s
        for p in range(2):
            pltpu.sync_copy(zeros_hbm, accum.at[pl.ds(s * stripe, stripe)])
            pltpu.sync_copy(cnt_hbm.at[p * _SC_TILES + tid], cntv)
            plsc.subcore_barrier()
            nch = jnp.max(cntv[...])
            rowbase = (p * _SC_TILES + tid) * capg

            def chunkbody(ch, carry):
                gbase = rowbase + ch * nc
                pltpu.sync_copy(pdst_hbm.at[pl.ds(gbase, nc)], dstv)
                pltpu.sync_copy(psrc_hbm.at[pl.ds(gbase, nc)], srcv)
                for b in range(2):
                    pltpu.async_copy(y_hbm.at[srcv.at[b]], rows.at[b],
                                     gsem.at[b])

                def quadbody(i, cc):
                    for b in range(4):
                        j = 4 * i + b
                        bf = (b + 2) % 4

                        @pl.when(j < nc)
                        def _():
                            pltpu.make_async_copy(
                                y_hbm.at[srcv.at[0]], rows.at[b],
                                gsem.at[b]).wait()
                            pltpu.async_copy(rows.at[b], accum.at[dstv.at[j]],
                                             ssem.at[b], add=True)

                            @pl.when(j >= 2)
                            def _():
                                pltpu.make_async_copy(
                                    rows.at[bf], accum.at[dstv.at[0]],
                                    ssem.at[bf]).wait()

                            @pl.when(j + 2 < nc)
                            def _():
                                pltpu.async_copy(y_hbm.at[srcv.at[j + 2]],
                                                 rows.at[bf], gsem.at[bf])
                    return cc

                lax.fori_loop(0, (nc + 3) // 4, quadbody, 0)
                for jt in range(max(0, nc - 2), nc):
                    b = jt % 4
                    pltpu.make_async_copy(rows.at[b], accum.at[dstv.at[0]],
                                          ssem.at[b]).wait()
                return carry

            lax.fori_loop(0, nch, chunkbody, 0)
            plsc.subcore_barrier()
            pltpu.sync_copy(
                accum.at[pl.ds(s * stripe, stripe)],
                out_hbm.at[pl.ds(c * n + p * half + s * stripe, stripe)])
            if p == 0:
                plsc.subcore_barrier()

    return pl.kernel(
        body,
        out_type=jax.ShapeDtypeStruct((_SC_CORES * n, w), jnp.float32),
        mesh=mesh,
        scratch_types=[
            pltpu.VMEM_SHARED((half_pad, w), jnp.float32),
            pltpu.VMEM((nc, g), jnp.int32),
            pltpu.VMEM((nc, g), jnp.int32),
            pltpu.VMEM((16,), jnp.int32),
            pltpu.VMEM((4, g, w), jnp.float32),
            pltpu.SemaphoreType.DMA((4,)),
            pltpu.SemaphoreType.DMA((4,)),
        ],
        compiler_params=pltpu.CompilerParams(use_tc_tiling_on_sc=False,
                                             needs_layout_passes=False),
    )


def _seg_partials_part(y, part, n):
    psrc, pdst, cnt, capg = part
    w = y.shape[1]
    zeros = jnp.zeros((n // 2 // _SC_SUBCORES, w), jnp.float32)
    kfn = _make_seg_part_kernel(n, w, capg)
    out = kfn(y, psrc, pdst, cnt, zeros)
    return out.reshape(_SC_CORES, n, w)


def _dst_split_body(d_ref, lo_ref, hi_ref, *, half):
    d = d_ref[...]
    lo_ref[...] = jnp.where(d < half, d, half)
    hi_ref[...] = jnp.where(d >= half, d - half, half)


def _dst_split(dst, half):
    e = dst.shape[0]
    rows = e // 128
    x = dst.reshape(rows, 128)
    br = 512
    body = functools.partial(_dst_split_body, half=half)
    lo, hi = pl.pallas_call(
        body,
        grid=(rows // br,),
        in_specs=[pl.BlockSpec((br, 128), lambda i: (i, 0))],
        out_specs=[pl.BlockSpec((br, 128), lambda i: (i, 0))] * 2,
        out_shape=[jax.ShapeDtypeStruct((rows, 128), jnp.int32)] * 2,
    )(x)
    return lo.reshape(e), hi.reshape(e)


_SPMEM_BUDGET = 7 * 1024 * 1024


def _num_passes(n, w):
    p = 1
    while (n // p + 128) * w * 4 > _SPMEM_BUDGET:
        p *= 2
    return p


def _sage(xs, src, dstls_fn, dp, params, name, n, relu=True, part=None):
    ws = params[name + "_Ws"]
    wn = params[name + "_Wn"]
    b = params[name + "_b"]
    ci, co = ws.shape
    w = min(ci, co)
    npass = _num_passes(n, w)
    offs = [0]
    for x in xs:
        offs.append(offs[-1] + x.shape[1])
    assert offs[-1] == ci
    wss = [ws[offs[i]:offs[i + 1]] for i in range(len(xs))]
    wns = [wn[offs[i]:offs[i + 1]] for i in range(len(xs))]

    colsplit = (npass > 1 and w % 2 == 0
                and (n + 128) * (w // 2) * 4 <= _SPMEM_BUDGET)

    def agg(y):
        if colsplit:
            return _seg_partials_colsplit(y, src, dstls_fn(1)[0], n)
        return _seg_partials(y, src, dstls_fn(npass), n)

    if co < ci:
        if len(xs) == 1:
            y = _mm(xs[0], wns[0])
        else:
            y = _mm2(xs[0], xs[1], wns[0], wns[1])
        p = agg(y)
        return _combine(xs, wss, None, p, dp, b, relu, colsplit)
    assert len(xs) == 1
    p = agg(xs[0])
    return _combine(xs, wss, wn, p, dp, b, relu, colsplit)


def kernel(in_feat, exteraVar1, params, edge_index1, edge_index2, edge_index3,
           edge_index4, edge_index5):
    del exteraVar1
    edges = [edge_index1, edge_index2, edge_index3, edge_index4, edge_index5]
    srcs = [e[0] for e in edges]
    dsts = [e[1] for e in edges]
    split_cache = {}

    def dstls_fn(lvl):
        def get(n_passes):
            if n_passes == 1:
                return [dsts[lvl]]
            key = (lvl, n_passes)
            if key not in split_cache:
                assert n_passes == 2
                split_cache[key] = list(_dst_split(dsts[lvl], _N[lvl] // 2))
            return split_cache[key]
        return get

    fns = [dstls_fn(l) for l in range(5)]
    dps = [_deg_partials(fns[l](_num_passes(_N[l], 16)), dsts[l].shape[0], _N[l])
           for l in range(5)]

    def sage(xs, lvl, name, relu=True):
        if not isinstance(xs, list):
            xs = [xs]
        return _sage(xs, srcs[lvl], fns[lvl], dps[lvl], params, name, _N[lvl],
                     relu, part=None)

    h1 = sage(in_feat, 0, "conv1")
    h22 = sage(h1, 0, "conv2")
    h2 = _pool(h22, RES)
    h3 = sage(h2, 1, "conv3")
    h33 = sage(h3, 1, "conv33")
    h3p = _pool(h33, RES // 2)
    h4 = sage(h3p, 2, "conv4")
    h44 = sage(h4, 2, "conv44")
    h4p = _pool(h44, RES // 4)
    h5 = sage(h4p, 3, "conv5")
    h55 = sage(h5, 3, "conv55")
    h5p = _pool(h55, RES // 8)
    h6 = sage(h5p, 4, "conv6")
    h6 = sage(h6, 4, "conv66")
    h6 = sage(h6, 4, "conv7")
    h6 = _up(h6, RES // 16, params["up1_W"], params["up1_b"])
    h6 = sage([h6, h55], 3, "conv7")
    h6 = sage(h6, 3, "conv77")
    h6 = sage(h6, 3, "conv8")
    h6 = _up(h6, RES // 8, params["up2_W"], params["up2_b"])
    h6 = sage([h6, h44], 2, "conv8")
    h6 = sage(h6, 2, "conv88")
    h6 = sage(h6, 2, "conv9")
    h6 = _up(h6, RES // 4, params["up3_W"], params["up3_b"])
    h6 = sage([h6, h33], 1, "conv9")
    h6 = sage(h6, 1, "conv99")
    h6 = sage(h6, 1, "conv10")
    h6 = _up(h6, RES // 2, params["up4_W"], params["up4_b"])
    h6 = sage([h6, h22], 0, "conv10")
    h6 = sage(h6, 0, "conv101")
    return sage(h6, 0, "conv11", relu=False)
